# Initial kernel scaffold; baseline (speedup 1.0000x reference)
import jax
import jax.numpy as jnp
from jax.experimental import pallas as pl


def kernel(h, edge_index, taxo_cats, node2item, W_w, W_b, prelu_w, taxo_mean, taxo_std_log, wh_w, convW_w, convW_b, psi_w, psi_b, mlp0_w, mlp0_b, mlp1_w, mlp1_b, mlp2_w, mlp2_b):
    raise NotImplementedError("write your pallas kernel here")


# speedup vs baseline: 7.9995x; 7.9995x over previous
import functools

import jax
import jax.numpy as jnp
from jax import lax
from jax.experimental import pallas as pl
from jax.experimental.pallas import tpu as pltpu
from jax.experimental.pallas import tpu_sc as plsc

N = 10000
E = 320000
M = 5000
T = 500
D = 128
H = 128
ETA = 0.5

NC = 2
NS = 16
NW = NC * NS

N_PAD = 10240
M_PAD = 5120
E_PAD = 327680
TK = 512
DUMP = 10200

EPW = E_PAD // NW
EPT = E_PAD // NS
OB = 2048
NB_A = EPT // OB
NB_B = EPW // OB
NSL = N_PAD // NS
MSL = M_PAD // NS

_SC_PARAMS = pltpu.CompilerParams(needs_layout_passes=False,
                                  use_tc_tiling_on_sc=False)
_mesh = plsc.VectorSubcoreMesh(core_axis_name="c", subcore_axis_name="s",
                               num_cores=NC, num_subcores=NS)

_HIGH = jax.lax.Precision.HIGHEST


def _f32(shape):
    return jax.ShapeDtypeStruct(shape, jnp.float32)


def _i32(shape):
    return jax.ShapeDtypeStruct(shape, jnp.int32)


def _iota16():
    return lax.broadcasted_iota(jnp.int32, (16,), 0)


def _tc_pre_body(h_ref, W_ref, b_ref, pw_ref, wh_ref, tm512_ref,
                 h0q0_ref, h0q1_ref, h0q2_ref, h0q3_ref, fsd_ref, K_ref, consts_ref):
    x = jnp.dot(h_ref[...], W_ref[...].T, precision=_HIGH,
                preferred_element_type=jnp.float32) + b_ref[...]
    pw = pw_ref[0, 0]
    h0 = jnp.where(x >= 0, x, pw * x)
    h0q0_ref[...] = h0[:, 0:32]
    h0q1_ref[...] = h0[:, 32:64]
    h0q2_ref[...] = h0[:, 64:96]
    h0q3_ref[...] = h0[:, 96:128]
    wh = wh_ref[...]
    fs = jnp.dot(h0, wh[0, :H], precision=_HIGH, preferred_element_type=jnp.float32)
    fd = jnp.dot(h0, wh[0, H:], precision=_HIGH, preferred_element_type=jnp.float32)
    fsd_ref[0, :] = fs
    fsd_ref[1, :] = fd
    K = jnp.dot(tm512_ref[...], tm512_ref[...].T, precision=_HIGH,
                preferred_element_type=jnp.float32)
    K_ref[...] = K
    cmax = jnp.max(fs) + jnp.max(fd)
    cf = jnp.where(cmax >= 0, cmax, 0.01 * cmax)
    ct = 2.0 * jnp.max(K)
    col = lax.broadcasted_iota(jnp.int32, (1, 128), 1)
    consts_ref[...] = jnp.where(col == 0, cf, jnp.where(col == 1, ct, 0.0))


def _tc_pre(h_p, W_w, W_b, prelu_w, wh_w, tm512):
    return pl.pallas_call(
        _tc_pre_body,
        out_shape=[_f32((N_PAD, 32))] * 4 + [_f32((2, N_PAD)),
                   _f32((TK, TK)), _f32((1, 128))],
    )(h_p, W_w, W_b.reshape(1, H), prelu_w.reshape(1, 1), wh_w, tm512)


def _tc_mid_body(z_ref, convW_ref, b_ref, pw_ref, wh_ref,
                 h1q0_ref, h1q1_ref, h1q2_ref, h1q3_ref, fsd_ref, consts_ref):
    z = z_ref[0] + z_ref[1]
    x = jnp.dot(z, convW_ref[...].T, precision=_HIGH,
                preferred_element_type=jnp.float32) + b_ref[...]
    pw = pw_ref[0, 0]
    h1 = jnp.where(x >= 0, x, pw * x)
    h1q0_ref[...] = h1[:, 0:32]
    h1q1_ref[...] = h1[:, 32:64]
    h1q2_ref[...] = h1[:, 64:96]
    h1q3_ref[...] = h1[:, 96:128]
    wh = wh_ref[...]
    fs = jnp.dot(h1, wh[0, :H], precision=_HIGH, preferred_element_type=jnp.float32)
    fd = jnp.dot(h1, wh[0, H:], precision=_HIGH, preferred_element_type=jnp.float32)
    fsd_ref[0, :] = fs
    fsd_ref[1, :] = fd
    cmax = jnp.max(fs) + jnp.max(fd)
    cf = jnp.where(cmax >= 0, cmax, 0.01 * cmax)
    col = lax.broadcasted_iota(jnp.int32, (1, 128), 1)
    consts_ref[...] = jnp.where(col == 0, cf, 0.0)


def _tc_mid(z2, convW_w, convW_b, prelu_w, wh_w):
    return pl.pallas_call(
        _tc_mid_body,
        out_shape=[_f32((N_PAD, 32))] * 4 + [_f32((2, N_PAD)), _f32((1, 128))],
    )(z2, convW_w, convW_b.reshape(1, H), prelu_w.reshape(1, 1), wh_w)


def _tc_raw_body(z_ref, convW_ref, b_ref, raw_ref):
    z = z_ref[0] + z_ref[1]
    raw_ref[...] = jnp.dot(z, convW_ref[...].T, precision=_HIGH,
                           preferred_element_type=jnp.float32) + b_ref[...]


def _tc_raw(z2, convW_w, convW_b):
    return pl.pallas_call(
        _tc_raw_body, out_shape=_f32((N_PAD, D)),
    )(z2, convW_w, convW_b.reshape(1, H))


def _tc_post_body(hrev_ref, psi_ref, psib_ref, tm0_ref, tm1_ref, sl0_ref, sl1_ref,
                  cur0_ref, cur1_ref):
    hrev = hrev_ref[0] + hrev_ref[1]
    hp = jnp.dot(hrev, psi_ref[...], precision=_HIGH,
                 preferred_element_type=jnp.float32)
    pb = psib_ref[0, 0]
    for tm_ref, sl_ref, cur_ref in ((tm0_ref, sl0_ref, cur0_ref),
                                    (tm1_ref, sl1_ref, cur1_ref)):
        tm = tm_ref[...]
        stpl = jax.nn.sigmoid(jnp.sum(hp * tm, axis=1, keepdims=True) + pb)
        tau = stpl * jnp.exp(-jnp.exp(sl_ref[...]))
        cur_ref[...] = (1.0 - tau) * hrev + tau * tm


def _tc_post(hrev_p, psi0, psi_b, tm0, tm1, sl0, sl1):
    return pl.pallas_call(
        _tc_post_body,
        out_shape=[_f32((M_PAD, D)), _f32((M_PAD, D))],
    )(hrev_p, psi0, psi_b.reshape(1, 1), tm0, tm1, sl0, sl1)


def _l2norm(y):
    n = jnp.sqrt(jnp.sum(y * y, axis=1, keepdims=True))
    return y / jnp.maximum(n, 1e-12)


def _tc_emb_body(raw_ref, c0_ref, c1_ref, w2_ref, b2_ref, w0_ref, b0_ref,
                 w1_ref, b1_ref, eh_ref, e0_ref, e1_ref):
    eh_ref[...] = _l2norm(jnp.dot(raw_ref[...], w2_ref[...].T, precision=_HIGH,
                                  preferred_element_type=jnp.float32) + b2_ref[...])
    e0_ref[...] = _l2norm(jnp.dot(c0_ref[...], w0_ref[...].T, precision=_HIGH,
                                  preferred_element_type=jnp.float32) + b0_ref[...])
    e1_ref[...] = _l2norm(jnp.dot(c1_ref[...], w1_ref[...].T, precision=_HIGH,
                                  preferred_element_type=jnp.float32) + b1_ref[...])


def _tc_emb(raw, cur0n, cur1n, mlp2_w, mlp2_b, mlp0_w, mlp0_b, mlp1_w, mlp1_b):
    k = mlp2_w.shape[0]
    return pl.pallas_call(
        _tc_emb_body,
        out_shape=[_f32((N_PAD, k))] * 3,
        compiler_params=pltpu.CompilerParams(vmem_limit_bytes=100 * 1024 * 1024),
    )(raw, cur0n, cur1n, mlp2_w, mlp2_b.reshape(1, k), mlp0_w, mlp0_b.reshape(1, k),
      mlp1_w, mlp1_b.reshape(1, k))


def _wid():
    return lax.axis_index("c") * NS + lax.axis_index("s")


def _zero_buf(buf, n):
    zeros = jnp.zeros((16,), jnp.float32)

    def body(i, _):
        buf[pl.ds(i * 16, 16)] = zeros
        return 0

    lax.fori_loop(0, n // 16, body, 0)


def _sc_prep_body(n2i_ref, tc_ref, tm_ref, sl_ref,
                  c0n_ref, c1n_ref, tm0_ref, tm1_ref, sl0_ref, sl1_ref,
                  tc_tab, nbuf, c0buf, c1buf, ibuf, rows):
    c = lax.axis_index("c")
    s = lax.axis_index("s")
    w = c * NS + s
    pltpu.sync_copy(tc_ref, tc_tab)
    nbase = w * (N_PAD // NW)
    pltpu.sync_copy(n2i_ref.at[pl.ds(nbase, 320)], nbuf)

    def nbody(i, _):
        it = nbuf[pl.ds(i * 16, 16)]
        c0buf[pl.ds(i * 16, 16)] = plsc.load_gather(tc_tab, [it * 2])
        c1buf[pl.ds(i * 16, 16)] = plsc.load_gather(tc_tab, [it * 2 + 1])
        return 0

    lax.fori_loop(0, 20, nbody, 0)
    pltpu.sync_copy(c0buf, c0n_ref.at[pl.ds(nbase, 320)])
    pltpu.sync_copy(c1buf, c1n_ref.at[pl.ds(nbase, 320)])

    mbase = w * (M_PAD // NW)
    iota = _iota16()
    for lidx, outs in ((0, (tm0_ref, sl0_ref)), (1, (tm1_ref, sl1_ref))):
        def ibody(i, _, _l=lidx):
            item = mbase + i * 16 + iota
            ibuf[pl.ds(i * 16, 16)] = plsc.load_gather(tc_tab, [item * 2 + _l])
            return 0

        lax.fori_loop(0, 10, ibody, 0)
        for tab_ref, out_ref in ((tm_ref, outs[0]), (sl_ref, outs[1])):
            pltpu.sync_copy(tab_ref.at[ibuf.at[pl.ds(0, 128)]], rows.at[pl.ds(0, 128)])
            pltpu.sync_copy(tab_ref.at[ibuf.at[pl.ds(128, 32)]], rows.at[pl.ds(128, 32)])
            pltpu.sync_copy(rows, out_ref.at[pl.ds(mbase, 160)])


def _sc_prep(n2i_p, tc_flat, taxo_mean, taxo_std_log):
    f = pl.kernel(
        _sc_prep_body,
        out_type=[_i32((N_PAD,)), _i32((N_PAD,)),
                  _f32((M_PAD, D)), _f32((M_PAD, D)),
                  _f32((M_PAD, D)), _f32((M_PAD, D))],
        mesh=_mesh,
        compiler_params=_SC_PARAMS,
        scratch_types=[
            pltpu.VMEM((2 * M_PAD,), jnp.int32),
            pltpu.VMEM((320,), jnp.int32),
            pltpu.VMEM((320,), jnp.int32),
            pltpu.VMEM((320,), jnp.int32),
            pltpu.VMEM((160,), jnp.int32),
            pltpu.VMEM((160, D), jnp.float32),
        ],
    )
    return f(n2i_p, tc_flat, taxo_mean, taxo_std_log)


def _fill_dst2d(dst2d, i, dv):
    row = i // 8
    col = (i % 8) * 16
    dst2d[row, pl.ds(col, 16)] = dv


def _sc_alphat_body(src_ref, dst_ref, c0n_ref, c1n_ref, kf_ref,
                    et_ref, sumtp_ref, mxall_ref,
                    c0_tab, c1_tab, mx_tab, wt_store, sbuf, dbuf,
                    ka, kb, kva, kvb, etbuf, dst2d, zbuf, macc, mtmp,
                    msegm_sh, sumt_sh):
    c = lax.axis_index("c")
    s = lax.axis_index("s")
    pltpu.sync_copy(c0n_ref, c0_tab)
    pltpu.sync_copy(c1n_ref, c1_tab)
    neg = jnp.full((16,), -1e30, jnp.float32)

    def initm(i, _):
        mx_tab[pl.ds(i * 16, 16)] = neg
        return 0

    lax.fori_loop(0, N_PAD // 16, initm, 0)
    _zero_buf(zbuf, NSL)
    pltpu.sync_copy(zbuf, sumt_sh.at[pl.ds(s * NSL, NSL)])

    abase = s * EPT
    for ob in range(NB_A):
        b0 = abase + ob * OB
        pltpu.sync_copy(src_ref.at[pl.ds(b0, OB)], sbuf)
        pltpu.sync_copy(dst_ref.at[pl.ds(b0, OB)], dbuf)

        def body1(i, _):
            sv = sbuf[pl.ds(i * 16, 16)]
            dv = dbuf[pl.ds(i * 16, 16)]
            a0 = plsc.load_gather(c0_tab, [sv])
            b0v = plsc.load_gather(c0_tab, [dv])
            a1 = plsc.load_gather(c1_tab, [sv])
            b1v = plsc.load_gather(c1_tab, [dv])
            ka[pl.ds(i * 16, 16)] = a0 * TK + b0v
            kb[pl.ds(i * 16, 16)] = a1 * TK + b1v
            return 0

        lax.fori_loop(0, OB // 16, body1, 0)

        def body2(j, _):
            pltpu.sync_copy(kf_ref.at[ka.at[pl.ds(j * 128, 128)]],
                            kva.at[pl.ds(j * 128, 128)])
            pltpu.sync_copy(kf_ref.at[kb.at[pl.ds(j * 128, 128)]],
                            kvb.at[pl.ds(j * 128, 128)])
            return 0

        lax.fori_loop(0, OB // 128, body2, 0)

        def body3(i, _, _ob=ob):
            dv = dbuf[pl.ds(i * 16, 16)]
            wt = kva[pl.ds(i * 16, 16)] + kvb[pl.ds(i * 16, 16)]
            wt_store[pl.ds(_ob * OB + i * 16, 16)] = wt
            cur = plsc.load_gather(mx_tab, [dv])
            need = wt > cur

            def mbody(m):
                plsc.store_scatter(mx_tab, [dv], wt, mask=m)
                cur2 = plsc.load_gather(mx_tab, [dv])
                return wt > cur2

            lax.while_loop(lambda m: jnp.any(m), mbody, need)
            return 0

        lax.fori_loop(0, OB // 16, body3, 0)

    pltpu.sync_copy(mx_tab, mxall_ref.at[c, s])
    plsc.subcore_barrier()
    pltpu.sync_copy(mxall_ref.at[c, 0, pl.ds(s * NSL, NSL)], macc)
    for t in range(1, NS):
        pltpu.sync_copy(mxall_ref.at[c, t, pl.ds(s * NSL, NSL)], mtmp)

        def mergeb(i, _):
            macc[pl.ds(i * 16, 16)] = jnp.maximum(macc[pl.ds(i * 16, 16)],
                                                  mtmp[pl.ds(i * 16, 16)])
            return 0

        lax.fori_loop(0, NSL // 16, mergeb, 0)
    pltpu.sync_copy(macc, msegm_sh.at[pl.ds(s * NSL, NSL)])
    plsc.subcore_barrier()
    pltpu.sync_copy(msegm_sh, mx_tab)

    bbase = s * EPT + c * EPW
    loff = c * EPW
    for ob in range(NB_B):
        b0 = bbase + ob * OB
        lo = loff + ob * OB
        pltpu.sync_copy(dst_ref.at[pl.ds(b0, OB)], dbuf)

        def body4(i, _, _lo=lo):
            dv = dbuf[pl.ds(i * 16, 16)]
            ms = plsc.load_gather(mx_tab, [dv])
            wt = wt_store[pl.ds(_lo + i * 16, 16)]
            etv = jnp.exp(wt - ms)
            etbuf[pl.ds(i * 16, 16)] = etv
            _fill_dst2d(dst2d, i, dv)
            return 0

        lax.fori_loop(0, OB // 16, body4, 0)
        pltpu.sync_copy(etbuf, et_ref.at[pl.ds(b0, OB)])

        def body5(j, _):
            pltpu.sync_copy(etbuf.at[pl.ds(j * 128, 128)],
                            sumt_sh.at[dst2d.at[j]], add=True)
            return 0

        lax.fori_loop(0, OB // 128, body5, 0)

    plsc.subcore_barrier()
    pltpu.sync_copy(sumt_sh.at[pl.ds(s * NSL, NSL)], zbuf)
    pltpu.sync_copy(zbuf, sumtp_ref.at[c, pl.ds(s * NSL, NSL)])


def _sc_alphat(src_p, dst_p, c0n, c1n, kflat):
    f = pl.kernel(
        _sc_alphat_body,
        out_type=[_f32((E_PAD,)), _f32((NC, N_PAD)), _f32((NC, NS, N_PAD))],
        mesh=_mesh,
        compiler_params=_SC_PARAMS,
        scratch_types=[
            pltpu.VMEM((N_PAD,), jnp.int32),
            pltpu.VMEM((N_PAD,), jnp.int32),
            pltpu.VMEM((N_PAD,), jnp.float32),
            pltpu.VMEM((EPT,), jnp.float32),
            pltpu.VMEM((OB,), jnp.int32),
            pltpu.VMEM((OB,), jnp.int32),
            pltpu.VMEM((OB,), jnp.int32),
            pltpu.VMEM((OB,), jnp.int32),
            pltpu.VMEM((OB,), jnp.float32),
            pltpu.VMEM((OB,), jnp.float32),
            pltpu.VMEM((OB,), jnp.float32),
            pltpu.VMEM((16, 128), jnp.int32),
            pltpu.VMEM((NSL,), jnp.float32),
            pltpu.VMEM((NSL,), jnp.float32),
            pltpu.VMEM((NSL,), jnp.float32),
            pltpu.VMEM_SHARED((N_PAD,), jnp.float32),
            pltpu.VMEM_SHARED((N_PAD,), jnp.float32),
        ],
    )
    return f(src_p, dst_p, c0n, c1n, kflat)


def _sc_layer_body(src_ref, dst_ref, fsd_ref, et_ref, sumtp_ref,
                   hq0_ref, hq1_ref, hq2_ref, hq3_ref,
                   consts_ref, zout_ref,
                   fs_tab, fd_tab, sumf_tab, sumt_tab, ef_store,
                   sbuf, dbuf, etbuf, abuf, dst2d, dst2d_full, rows, zbuf, cbuf,
                   sumf_sh, z_sh):
    c = lax.axis_index("c")
    s = lax.axis_index("s")
    pltpu.sync_copy(fsd_ref.at[0], fs_tab)
    pltpu.sync_copy(fsd_ref.at[1], fd_tab)
    pltpu.sync_copy(consts_ref.at[pl.ds(0, 16)], cbuf)
    pltpu.sync_copy(sumtp_ref.at[0], sumt_tab)
    pltpu.sync_copy(sumtp_ref.at[1], sumf_tab)

    def addb(i, _):
        sumt_tab[pl.ds(i * 16, 16)] = (sumt_tab[pl.ds(i * 16, 16)]
                                       + sumf_tab[pl.ds(i * 16, 16)])
        return 0

    lax.fori_loop(0, N_PAD // 16, addb, 0)
    _zero_buf(zbuf, NSL)
    pltpu.sync_copy(zbuf, sumf_sh.at[pl.ds(s * NSL, NSL)])
    plsc.subcore_barrier()

    cf = cbuf[pl.ds(0, 16)][0]
    abase = s * EPT
    for ob in range(NB_A):
        b0 = abase + ob * OB
        pltpu.sync_copy(src_ref.at[pl.ds(b0, OB)], sbuf)
        pltpu.sync_copy(dst_ref.at[pl.ds(b0, OB)], dbuf)

        def bodyA(i, _, _ob=ob):
            sv = sbuf[pl.ds(i * 16, 16)]
            dv = dbuf[pl.ds(i * 16, 16)]
            x = (plsc.load_gather(fs_tab, [sv])
                 + plsc.load_gather(fd_tab, [dv]))
            wf = jnp.where(x >= 0, x, 0.01 * x)
            ef = jnp.exp(wf - cf)
            ef_store[pl.ds(_ob * OB + i * 16, 16)] = ef
            _fill_dst2d(dst2d, i, dv)
            return 0

        lax.fori_loop(0, OB // 16, bodyA, 0)

        def bodyA2(j, _, _ob=ob):
            pltpu.sync_copy(ef_store.at[pl.ds(_ob * OB + j * 128, 128)],
                            sumf_sh.at[dst2d.at[j]], add=True)
            return 0

        lax.fori_loop(0, OB // 128, bodyA2, 0)

    plsc.subcore_barrier()
    pltpu.sync_copy(sumf_sh, sumf_tab)

    bbase = s * EPT + c * EPW
    loff = c * EPW
    for ob in range(NB_B):
        b0 = bbase + ob * OB
        lo = loff + ob * OB
        pltpu.sync_copy(dst_ref.at[pl.ds(b0, OB)], dbuf)
        pltpu.sync_copy(et_ref.at[pl.ds(b0, OB)], etbuf)

        def bodyB1(i, _, _lo=lo, _ob=ob):
            dv = dbuf[pl.ds(i * 16, 16)]
            sf = jnp.maximum(plsc.load_gather(sumf_tab, [dv]), 1e-30)
            st = jnp.maximum(plsc.load_gather(sumt_tab, [dv]), 1e-30)
            ef = ef_store[pl.ds(_lo + i * 16, 16)]
            etv = etbuf[pl.ds(i * 16, 16)]
            abuf[pl.ds(_ob * OB + i * 16, 16)] = ETA * ef / sf + (1.0 - ETA) * etv / st
            row = _ob * 16 + i // 8
            col = (i % 8) * 16
            dst2d_full[row, pl.ds(col, 16)] = dv
            return 0

        lax.fori_loop(0, OB // 16, bodyB1, 0)

    zeros16 = jnp.zeros((16,), jnp.float32)
    for q, hin_ref in ((0, hq0_ref), (1, hq1_ref), (2, hq2_ref), (3, hq3_ref)):
        def zrows(r, _):
            for k in range(2):
                rows[r, pl.ds(k * 16, 16)] = zeros16
            return 0

        lax.fori_loop(0, 128, zrows, 0)

        def zcopy(t, _):
            pltpu.sync_copy(rows, z_sh.at[pl.ds(s * NSL + t * 128, 128)])
            return 0

        lax.fori_loop(0, NSL // 128, zcopy, 0)
        plsc.subcore_barrier()

        for ob in range(NB_B):
            b0 = bbase + ob * OB
            pltpu.sync_copy(src_ref.at[pl.ds(b0, OB)], sbuf)

            def bodyB2(j, _, _ob=ob, _hin=hin_ref):
                pltpu.sync_copy(_hin.at[sbuf.at[pl.ds(j * 128, 128)]], rows)

                def scale(r, _, _j=j, _ob2=_ob):
                    a = abuf[pl.ds(_ob2 * OB + _j * 128 + r, 16)][0]
                    for k in range(2):
                        rows[r, pl.ds(k * 16, 16)] = rows[r, pl.ds(k * 16, 16)] * a
                    return 0

                lax.fori_loop(0, 128, scale, 0)
                pltpu.sync_copy(rows, z_sh.at[dst2d_full.at[_ob * 16 + j]], add=True)
                return 0

            lax.fori_loop(0, OB // 128, bodyB2, 0)

        plsc.subcore_barrier()

        def outb(t, _, _q=q):
            pltpu.sync_copy(z_sh.at[pl.ds(s * NSL + t * 128, 128)], rows)
            pltpu.sync_copy(rows, zout_ref.at[c, pl.ds(s * NSL + t * 128, 128),
                                              pl.ds(_q * 32, 32)])
            return 0

        lax.fori_loop(0, NSL // 128, outb, 0)
        plsc.subcore_barrier()


def _sc_layer(src_p, dst_p, fsd, et, sumt_p, hq0, hq1, hq2, hq3, consts):
    f = pl.kernel(
        _sc_layer_body,
        out_type=_f32((NC, N_PAD, D)),
        mesh=_mesh,
        compiler_params=_SC_PARAMS,
        scratch_types=[
            pltpu.VMEM((N_PAD,), jnp.float32),
            pltpu.VMEM((N_PAD,), jnp.float32),
            pltpu.VMEM((N_PAD,), jnp.float32),
            pltpu.VMEM((N_PAD,), jnp.float32),
            pltpu.VMEM((EPT,), jnp.float32),
            pltpu.VMEM((OB,), jnp.int32),
            pltpu.VMEM((OB,), jnp.int32),
            pltpu.VMEM((OB,), jnp.float32),
            pltpu.VMEM((EPW + 16,), jnp.float32),
            pltpu.VMEM((16, 128), jnp.int32),
            pltpu.VMEM((NB_B * 16, 128), jnp.int32),
            pltpu.VMEM((128, 32), jnp.float32),
            pltpu.VMEM((NSL,), jnp.float32),
            pltpu.VMEM((16,), jnp.float32),
            pltpu.VMEM_SHARED((N_PAD,), jnp.float32),
            pltpu.VMEM_SHARED((N_PAD, 32), jnp.float32),
        ],
    )
    return f(src_p, dst_p, fsd, et, sumt_p, hq0, hq1, hq2, hq3, consts)


def _sc_hrev_body(hin_ref, n2i_ref, hrevp_ref, ibuf2d, rows, hrev_sh):
    c = lax.axis_index("c")
    s = lax.axis_index("s")
    w = c * NS + s
    zeros16 = jnp.zeros((16,), jnp.float32)
    base = s * MSL
    for half in (0, 1):
        def zrows(r, _):
            for k in range(4):
                rows[r, pl.ds(k * 16, 16)] = zeros16
            return 0

        lax.fori_loop(0, 128, zrows, 0)
        pltpu.sync_copy(rows, hrev_sh.at[pl.ds(base, 128)])
        pltpu.sync_copy(rows, hrev_sh.at[pl.ds(base + 128, 128)])
        pltpu.sync_copy(rows.at[pl.ds(0, 64)], hrev_sh.at[pl.ds(base + 256, 64)])
        plsc.subcore_barrier()

        def chunk(ch, _half=half):
            r0 = ch * 128
            pltpu.sync_copy(n2i_ref.at[pl.ds(r0, 128)], ibuf2d.at[0])
            pltpu.sync_copy(hin_ref.at[pl.ds(r0, 128), pl.ds(_half * 64, 64)], rows)
            pltpu.sync_copy(rows, hrev_sh.at[ibuf2d.at[0]], add=True)

        chunk(w)
        chunk(w + 32)

        @pl.when(w < 16)
        def _():
            chunk(w + 64)

        plsc.subcore_barrier()
        for off, ln in ((0, 128), (128, 128), (256, 64)):
            pltpu.sync_copy(hrev_sh.at[pl.ds(base + off, ln)], rows.at[pl.ds(0, ln)])
            pltpu.sync_copy(rows.at[pl.ds(0, ln)],
                            hrevp_ref.at[c, pl.ds(base + off, ln), pl.ds(half * 64, 64)])
        plsc.subcore_barrier()


def _sc_hrev(hin, n2i_p):
    f = pl.kernel(
        _sc_hrev_body,
        out_type=_f32((NC, M_PAD, D)),
        mesh=_mesh,
        compiler_params=_SC_PARAMS,
        scratch_types=[
            pltpu.VMEM((1, 128), jnp.int32),
            pltpu.VMEM((128, 64), jnp.float32),
            pltpu.VMEM_SHARED((M_PAD, 64), jnp.float32),
        ],
    )
    return f(hin, n2i_p)


def _sc_gather_body(cur0_ref, cur1_ref, n2i_ref, out0_ref, out1_ref, ibuf, rows):
    c = lax.axis_index("c")
    s = lax.axis_index("s")
    w = c * NS + s
    nbase = w * (N_PAD // NW)
    pltpu.sync_copy(n2i_ref.at[pl.ds(nbase, 320)], ibuf)
    for tab_ref, out_ref in ((cur0_ref, out0_ref), (cur1_ref, out1_ref)):
        for off, ln in ((0, 128), (128, 128), (256, 64)):
            pltpu.sync_copy(tab_ref.at[ibuf.at[pl.ds(off, ln)]], rows.at[pl.ds(0, ln)])
            pltpu.sync_copy(rows.at[pl.ds(0, ln)], out_ref.at[pl.ds(nbase + off, ln)])


def _sc_gather(cur0, cur1, n2i_p):
    f = pl.kernel(
        _sc_gather_body,
        out_type=[_f32((N_PAD, D)), _f32((N_PAD, D))],
        mesh=_mesh,
        compiler_params=_SC_PARAMS,
        scratch_types=[
            pltpu.VMEM((320,), jnp.int32),
            pltpu.VMEM((128, D), jnp.float32),
        ],
    )
    return f(cur0, cur1, n2i_p)


def kernel(h, edge_index, taxo_cats, node2item, W_w, W_b, prelu_w, taxo_mean,
           taxo_std_log, wh_w, convW_w, convW_b, psi_w, psi_b, mlp0_w, mlp0_b,
           mlp1_w, mlp1_b, mlp2_w, mlp2_b):
    src_p = jnp.concatenate([edge_index[0], jnp.full((E_PAD - E,), DUMP, jnp.int32)])
    dst_p = jnp.concatenate([edge_index[1], jnp.full((E_PAD - E,), DUMP, jnp.int32)])
    n2i_p = jnp.concatenate([node2item.astype(jnp.int32),
                             jnp.full((N_PAD - N,), M, jnp.int32)])
    tc_flat = jnp.pad(taxo_cats.astype(jnp.int32), ((0, M_PAD - M), (0, 0))).reshape(-1)
    h_p = jnp.pad(h, ((0, N_PAD - N), (0, 0)))
    tm512 = jnp.pad(taxo_mean, ((0, TK - T), (0, 0)))

    h0q0, h0q1, h0q2, h0q3, fsd0, K, consts0 = _tc_pre(h_p, W_w, W_b, prelu_w,
                                                       wh_w, tm512)
    kflat = K.reshape(-1)
    consts0 = consts0.reshape(-1)

    c0n, c1n, tm0, tm1, sl0, sl1 = _sc_prep(n2i_p, tc_flat, taxo_mean, taxo_std_log)
    et, sumt_p, _mx = _sc_alphat(src_p, dst_p, c0n, c1n, kflat)

    z1 = _sc_layer(src_p, dst_p, fsd0, et, sumt_p, h0q0, h0q1, h0q2, h0q3, consts0)
    h1q0, h1q1, h1q2, h1q3, fsd1, consts1 = _tc_mid(z1, convW_w, convW_b,
                                                    prelu_w, wh_w)
    consts1 = consts1.reshape(-1)

    z2 = _sc_layer(src_p, dst_p, fsd1, et, sumt_p, h1q0, h1q1, h1q2, h1q3, consts1)
    raw = _tc_raw(z2, convW_w, convW_b)

    hrev_p = _sc_hrev(raw, n2i_p)
    cur0, cur1 = _tc_post(hrev_p, psi_w[0], psi_b, tm0, tm1, sl0, sl1)
    cur0n, cur1n = _sc_gather(cur0, cur1, n2i_p)
    eh, e0, e1 = _tc_emb(raw, cur0n, cur1n, mlp2_w, mlp2_b, mlp0_w, mlp0_b,
                         mlp1_w, mlp1_b)

    emb = jnp.concatenate(
        [eh[:N], jnp.stack([e0[:N], e1[:N]], axis=2).reshape(N, -1)], axis=1)
    return (raw[:N], emb)


# speedup vs baseline: 11.6983x; 1.4624x over previous
import functools

import jax
import jax.numpy as jnp
from jax import lax
from jax.experimental import pallas as pl
from jax.experimental.pallas import tpu as pltpu
from jax.experimental.pallas import tpu_sc as plsc

N = 10000
E = 320000
M = 5000
T = 500
D = 128
H = 128
ETA = 0.5

NC = 2
NS = 16
NW = NC * NS

N_PAD = 10240
M_PAD = 5120
E_PAD = 327680
TK = 512
DUMP = 10200

EPW = E_PAD // NW
EPT = E_PAD // NS
OB = 2048
NB_A = EPT // OB
NB_B = EPW // OB
NSL = N_PAD // NS
MSL = M_PAD // NS

_SC_PARAMS = pltpu.CompilerParams(needs_layout_passes=False,
                                  use_tc_tiling_on_sc=False)
_mesh = plsc.VectorSubcoreMesh(core_axis_name="c", subcore_axis_name="s",
                               num_cores=NC, num_subcores=NS)

_HIGH = jax.lax.Precision.HIGHEST


def _f32(shape):
    return jax.ShapeDtypeStruct(shape, jnp.float32)


def _i32(shape):
    return jax.ShapeDtypeStruct(shape, jnp.int32)


def _iota16():
    return lax.broadcasted_iota(jnp.int32, (16,), 0)


def _tc_pre_body(h_ref, W_ref, b_ref, pw_ref, wh_ref, tm512_ref,
                 h0q0_ref, h0q1_ref, h0q2_ref, h0q3_ref, fsd_ref, K_ref, consts_ref):
    x = jnp.dot(h_ref[...], W_ref[...].T, precision=_HIGH,
                preferred_element_type=jnp.float32) + b_ref[...]
    pw = pw_ref[0, 0]
    h0 = jnp.where(x >= 0, x, pw * x)
    h0q0_ref[...] = h0[:, 0:32]
    h0q1_ref[...] = h0[:, 32:64]
    h0q2_ref[...] = h0[:, 64:96]
    h0q3_ref[...] = h0[:, 96:128]
    wh = wh_ref[...]
    fs = jnp.dot(h0, wh[0, :H], precision=_HIGH, preferred_element_type=jnp.float32)
    fd = jnp.dot(h0, wh[0, H:], precision=_HIGH, preferred_element_type=jnp.float32)
    fsd_ref[0, :] = fs
    fsd_ref[1, :] = fd
    K = jnp.dot(tm512_ref[...], tm512_ref[...].T, precision=_HIGH,
                preferred_element_type=jnp.float32)
    K_ref[...] = K
    cmax = jnp.max(fs) + jnp.max(fd)
    cf = jnp.where(cmax >= 0, cmax, 0.01 * cmax)
    ct = 2.0 * jnp.max(K)
    col = lax.broadcasted_iota(jnp.int32, (1, 128), 1)
    consts_ref[...] = jnp.where(col == 0, cf, jnp.where(col == 1, ct, 0.0))


def _tc_pre(h_p, W_w, W_b, prelu_w, wh_w, tm512):
    return pl.pallas_call(
        _tc_pre_body,
        out_shape=[_f32((N_PAD, 32))] * 4 + [_f32((2, N_PAD)),
                   _f32((TK, TK)), _f32((1, 128))],
    )(h_p, W_w, W_b.reshape(1, H), prelu_w.reshape(1, 1), wh_w, tm512)


def _tc_mid_body(z_ref, convW_ref, b_ref, pw_ref, wh_ref,
                 h1q0_ref, h1q1_ref, h1q2_ref, h1q3_ref, fsd_ref, consts_ref):
    z = z_ref[0] + z_ref[1]
    x = jnp.dot(z, convW_ref[...].T, precision=_HIGH,
                preferred_element_type=jnp.float32) + b_ref[...]
    pw = pw_ref[0, 0]
    h1 = jnp.where(x >= 0, x, pw * x)
    h1q0_ref[...] = h1[:, 0:32]
    h1q1_ref[...] = h1[:, 32:64]
    h1q2_ref[...] = h1[:, 64:96]
    h1q3_ref[...] = h1[:, 96:128]
    wh = wh_ref[...]
    fs = jnp.dot(h1, wh[0, :H], precision=_HIGH, preferred_element_type=jnp.float32)
    fd = jnp.dot(h1, wh[0, H:], precision=_HIGH, preferred_element_type=jnp.float32)
    fsd_ref[0, :] = fs
    fsd_ref[1, :] = fd
    cmax = jnp.max(fs) + jnp.max(fd)
    cf = jnp.where(cmax >= 0, cmax, 0.01 * cmax)
    col = lax.broadcasted_iota(jnp.int32, (1, 128), 1)
    consts_ref[...] = jnp.where(col == 0, cf, 0.0)


def _tc_mid(z2, convW_w, convW_b, prelu_w, wh_w):
    return pl.pallas_call(
        _tc_mid_body,
        out_shape=[_f32((N_PAD, 32))] * 4 + [_f32((2, N_PAD)), _f32((1, 128))],
    )(z2, convW_w, convW_b.reshape(1, H), prelu_w.reshape(1, 1), wh_w)


def _tc_raw_body(z_ref, convW_ref, b_ref, raw_ref):
    z = z_ref[0] + z_ref[1]
    raw_ref[...] = jnp.dot(z, convW_ref[...].T, precision=_HIGH,
                           preferred_element_type=jnp.float32) + b_ref[...]


def _tc_raw(z2, convW_w, convW_b):
    return pl.pallas_call(
        _tc_raw_body, out_shape=_f32((N_PAD, D)),
    )(z2, convW_w, convW_b.reshape(1, H))


def _tc_post_body(hrev_ref, psi_ref, psib_ref, tm0_ref, tm1_ref, sl0_ref, sl1_ref,
                  cur0_ref, cur1_ref):
    hrev = hrev_ref[0] + hrev_ref[1]
    hp = jnp.dot(hrev, psi_ref[...], precision=_HIGH,
                 preferred_element_type=jnp.float32)
    pb = psib_ref[0, 0]
    for tm_ref, sl_ref, cur_ref in ((tm0_ref, sl0_ref, cur0_ref),
                                    (tm1_ref, sl1_ref, cur1_ref)):
        tm = tm_ref[...]
        stpl = jax.nn.sigmoid(jnp.sum(hp * tm, axis=1, keepdims=True) + pb)
        tau = stpl * jnp.exp(-jnp.exp(sl_ref[...]))
        cur_ref[...] = (1.0 - tau) * hrev + tau * tm


def _tc_post(hrev_p, psi0, psi_b, tm0, tm1, sl0, sl1):
    return pl.pallas_call(
        _tc_post_body,
        out_shape=[_f32((M_PAD, D)), _f32((M_PAD, D))],
    )(hrev_p, psi0, psi_b.reshape(1, 1), tm0, tm1, sl0, sl1)


def _l2norm(y):
    n = jnp.sqrt(jnp.sum(y * y, axis=1, keepdims=True))
    return y / jnp.maximum(n, 1e-12)


def _tc_emb_body(raw_ref, c0_ref, c1_ref, w2_ref, b2_ref, w0_ref, b0_ref,
                 w1_ref, b1_ref, eh_ref, e0_ref, e1_ref):
    eh_ref[...] = _l2norm(jnp.dot(raw_ref[...], w2_ref[...].T, precision=_HIGH,
                                  preferred_element_type=jnp.float32) + b2_ref[...])
    e0_ref[...] = _l2norm(jnp.dot(c0_ref[...], w0_ref[...].T, precision=_HIGH,
                                  preferred_element_type=jnp.float32) + b0_ref[...])
    e1_ref[...] = _l2norm(jnp.dot(c1_ref[...], w1_ref[...].T, precision=_HIGH,
                                  preferred_element_type=jnp.float32) + b1_ref[...])


def _tc_emb(raw, cur0n, cur1n, mlp2_w, mlp2_b, mlp0_w, mlp0_b, mlp1_w, mlp1_b):
    k = mlp2_w.shape[0]
    return pl.pallas_call(
        _tc_emb_body,
        out_shape=[_f32((N_PAD, k))] * 3,
        compiler_params=pltpu.CompilerParams(vmem_limit_bytes=100 * 1024 * 1024),
    )(raw, cur0n, cur1n, mlp2_w, mlp2_b.reshape(1, k), mlp0_w, mlp0_b.reshape(1, k),
      mlp1_w, mlp1_b.reshape(1, k))


def _wid():
    return lax.axis_index("c") * NS + lax.axis_index("s")


def _zero_buf(buf, n):
    zeros = jnp.zeros((16,), jnp.float32)

    def body(i, _):
        buf[pl.ds(i * 16, 16)] = zeros
        return 0

    lax.fori_loop(0, n // 16, body, 0)


def _sc_prep_body(n2i_ref, tc_ref, tm_ref, sl_ref,
                  c0n_ref, c1n_ref, tm0_ref, tm1_ref, sl0_ref, sl1_ref,
                  tc_tab, nbuf, c0buf, c1buf, ibuf, rows):
    c = lax.axis_index("c")
    s = lax.axis_index("s")
    w = c * NS + s
    pltpu.sync_copy(tc_ref, tc_tab)
    nbase = w * (N_PAD // NW)
    pltpu.sync_copy(n2i_ref.at[pl.ds(nbase, 320)], nbuf)

    def nbody(i, _):
        it = nbuf[pl.ds(i * 16, 16)]
        c0buf[pl.ds(i * 16, 16)] = plsc.load_gather(tc_tab, [it * 2])
        c1buf[pl.ds(i * 16, 16)] = plsc.load_gather(tc_tab, [it * 2 + 1])
        return 0

    lax.fori_loop(0, 20, nbody, 0)
    pltpu.sync_copy(c0buf, c0n_ref.at[pl.ds(nbase, 320)])
    pltpu.sync_copy(c1buf, c1n_ref.at[pl.ds(nbase, 320)])

    mbase = w * (M_PAD // NW)
    iota = _iota16()
    for lidx, outs in ((0, (tm0_ref, sl0_ref)), (1, (tm1_ref, sl1_ref))):
        def ibody(i, _, _l=lidx):
            item = mbase + i * 16 + iota
            ibuf[pl.ds(i * 16, 16)] = plsc.load_gather(tc_tab, [item * 2 + _l])
            return 0

        lax.fori_loop(0, 10, ibody, 0)
        for tab_ref, out_ref in ((tm_ref, outs[0]), (sl_ref, outs[1])):
            pltpu.sync_copy(tab_ref.at[ibuf.at[pl.ds(0, 128)]], rows.at[pl.ds(0, 128)])
            pltpu.sync_copy(tab_ref.at[ibuf.at[pl.ds(128, 32)]], rows.at[pl.ds(128, 32)])
            pltpu.sync_copy(rows, out_ref.at[pl.ds(mbase, 160)])


def _sc_prep(n2i_p, tc_flat, taxo_mean, taxo_std_log):
    f = pl.kernel(
        _sc_prep_body,
        out_type=[_i32((N_PAD,)), _i32((N_PAD,)),
                  _f32((M_PAD, D)), _f32((M_PAD, D)),
                  _f32((M_PAD, D)), _f32((M_PAD, D))],
        mesh=_mesh,
        compiler_params=_SC_PARAMS,
        scratch_types=[
            pltpu.VMEM((2 * M_PAD,), jnp.int32),
            pltpu.VMEM((320,), jnp.int32),
            pltpu.VMEM((320,), jnp.int32),
            pltpu.VMEM((320,), jnp.int32),
            pltpu.VMEM((160,), jnp.int32),
            pltpu.VMEM((160, D), jnp.float32),
        ],
    )
    return f(n2i_p, tc_flat, taxo_mean, taxo_std_log)


def _fill_dst2d(dst2d, i, dv):
    row = i // 8
    col = (i % 8) * 16
    dst2d[row, pl.ds(col, 16)] = dv


def _sc_alphat_body(src_ref, dst_ref, c0n_ref, c1n_ref, kf_ref,
                    et_ref, sumtp_ref, mxall_ref,
                    c0_tab, c1_tab, mx_tab, wt_store, sbuf, dbuf,
                    ka, kb, kva, kvb, etbuf, dst2d, zbuf, macc, mtmp,
                    msegm_sh, sumt_sh):
    c = lax.axis_index("c")
    s = lax.axis_index("s")
    pltpu.sync_copy(c0n_ref, c0_tab)
    pltpu.sync_copy(c1n_ref, c1_tab)
    neg = jnp.full((16,), -1e30, jnp.float32)

    def initm(i, _):
        mx_tab[pl.ds(i * 16, 16)] = neg
        return 0

    lax.fori_loop(0, N_PAD // 16, initm, 0)
    _zero_buf(zbuf, NSL)
    pltpu.sync_copy(zbuf, sumt_sh.at[pl.ds(s * NSL, NSL)])

    abase = s * EPT
    for ob in range(NB_A):
        b0 = abase + ob * OB
        pltpu.sync_copy(src_ref.at[pl.ds(b0, OB)], sbuf)
        pltpu.sync_copy(dst_ref.at[pl.ds(b0, OB)], dbuf)

        def body1(i, _):
            sv = sbuf[pl.ds(i * 16, 16)]
            dv = dbuf[pl.ds(i * 16, 16)]
            a0 = plsc.load_gather(c0_tab, [sv])
            b0v = plsc.load_gather(c0_tab, [dv])
            a1 = plsc.load_gather(c1_tab, [sv])
            b1v = plsc.load_gather(c1_tab, [dv])
            ka[pl.ds(i * 16, 16)] = a0 * TK + b0v
            kb[pl.ds(i * 16, 16)] = a1 * TK + b1v
            return 0

        lax.fori_loop(0, OB // 16, body1, 0)

        def body2(j, _):
            pltpu.sync_copy(kf_ref.at[ka.at[pl.ds(j * 128, 128)]],
                            kva.at[pl.ds(j * 128, 128)])
            pltpu.sync_copy(kf_ref.at[kb.at[pl.ds(j * 128, 128)]],
                            kvb.at[pl.ds(j * 128, 128)])
            return 0

        lax.fori_loop(0, OB // 128, body2, 0)

        def body3(i, _, _ob=ob):
            dv = dbuf[pl.ds(i * 16, 16)]
            wt = kva[pl.ds(i * 16, 16)] + kvb[pl.ds(i * 16, 16)]
            wt_store[pl.ds(_ob * OB + i * 16, 16)] = wt
            cur = plsc.load_gather(mx_tab, [dv])
            need = wt > cur

            def mbody(m):
                plsc.store_scatter(mx_tab, [dv], wt, mask=m)
                cur2 = plsc.load_gather(mx_tab, [dv])
                return wt > cur2

            lax.while_loop(lambda m: jnp.any(m), mbody, need)
            return 0

        lax.fori_loop(0, OB // 16, body3, 0)

    pltpu.sync_copy(mx_tab, mxall_ref.at[c, s])
    plsc.subcore_barrier()
    pltpu.sync_copy(mxall_ref.at[c, 0, pl.ds(s * NSL, NSL)], macc)
    for t in range(1, NS):
        pltpu.sync_copy(mxall_ref.at[c, t, pl.ds(s * NSL, NSL)], mtmp)

        def mergeb(i, _):
            macc[pl.ds(i * 16, 16)] = jnp.maximum(macc[pl.ds(i * 16, 16)],
                                                  mtmp[pl.ds(i * 16, 16)])
            return 0

        lax.fori_loop(0, NSL // 16, mergeb, 0)
    pltpu.sync_copy(macc, msegm_sh.at[pl.ds(s * NSL, NSL)])
    plsc.subcore_barrier()
    pltpu.sync_copy(msegm_sh, mx_tab)

    bbase = s * EPT + c * EPW
    loff = c * EPW
    for ob in range(NB_B):
        b0 = bbase + ob * OB
        lo = loff + ob * OB
        pltpu.sync_copy(dst_ref.at[pl.ds(b0, OB)], dbuf)

        def body4(i, _, _lo=lo):
            dv = dbuf[pl.ds(i * 16, 16)]
            ms = plsc.load_gather(mx_tab, [dv])
            wt = wt_store[pl.ds(_lo + i * 16, 16)]
            etv = jnp.exp(wt - ms)
            etbuf[pl.ds(i * 16, 16)] = etv
            _fill_dst2d(dst2d, i, dv)
            return 0

        lax.fori_loop(0, OB // 16, body4, 0)
        pltpu.sync_copy(etbuf, et_ref.at[pl.ds(b0, OB)])

        def body5(j, _):
            pltpu.sync_copy(etbuf.at[pl.ds(j * 128, 128)],
                            sumt_sh.at[dst2d.at[j]], add=True)
            return 0

        lax.fori_loop(0, OB // 128, body5, 0)

    plsc.subcore_barrier()
    pltpu.sync_copy(sumt_sh.at[pl.ds(s * NSL, NSL)], zbuf)
    pltpu.sync_copy(zbuf, sumtp_ref.at[c, pl.ds(s * NSL, NSL)])


def _sc_alphat(src_p, dst_p, c0n, c1n, kflat):
    f = pl.kernel(
        _sc_alphat_body,
        out_type=[_f32((E_PAD,)), _f32((NC, N_PAD)), _f32((NC, NS, N_PAD))],
        mesh=_mesh,
        compiler_params=_SC_PARAMS,
        scratch_types=[
            pltpu.VMEM((N_PAD,), jnp.int32),
            pltpu.VMEM((N_PAD,), jnp.int32),
            pltpu.VMEM((N_PAD,), jnp.float32),
            pltpu.VMEM((EPT,), jnp.float32),
            pltpu.VMEM((OB,), jnp.int32),
            pltpu.VMEM((OB,), jnp.int32),
            pltpu.VMEM((OB,), jnp.int32),
            pltpu.VMEM((OB,), jnp.int32),
            pltpu.VMEM((OB,), jnp.float32),
            pltpu.VMEM((OB,), jnp.float32),
            pltpu.VMEM((OB,), jnp.float32),
            pltpu.VMEM((16, 128), jnp.int32),
            pltpu.VMEM((NSL,), jnp.float32),
            pltpu.VMEM((NSL,), jnp.float32),
            pltpu.VMEM((NSL,), jnp.float32),
            pltpu.VMEM_SHARED((N_PAD,), jnp.float32),
            pltpu.VMEM_SHARED((N_PAD,), jnp.float32),
        ],
    )
    return f(src_p, dst_p, c0n, c1n, kflat)


def _sc_layer_body(src_ref, dst_ref, fsd_ref, et_ref, sumtp_ref,
                   hq0_ref, hq1_ref, hq2_ref, hq3_ref, consts_ref, zout_ref,
                   fs_tab, fd_tab, sumf_tab, sumt_tab,
                   sbuf_full, sbuf, dbuf, etbuf, abuf, dst2d, dst2d_full,
                   rows4, zbuf, cbuf,
                   g0, g1, g2, g3, s0, s1, s2, s3,
                   sumf_sh, z_sh):
    c = lax.axis_index("c")
    s = lax.axis_index("s")
    gsem = (g0, g1, g2, g3)
    ssem = (s0, s1, s2, s3)
    pltpu.sync_copy(fsd_ref.at[0], fs_tab)
    pltpu.sync_copy(fsd_ref.at[1], fd_tab)
    pltpu.sync_copy(consts_ref.at[pl.ds(0, 16)], cbuf)
    pltpu.sync_copy(sumtp_ref.at[0], sumt_tab)
    pltpu.sync_copy(sumtp_ref.at[1], sumf_tab)

    def addb(i, _):
        sumt_tab[pl.ds(i * 16, 16)] = (sumt_tab[pl.ds(i * 16, 16)]
                                       + sumf_tab[pl.ds(i * 16, 16)])
        return 0

    lax.fori_loop(0, N_PAD // 16, addb, 0)
    _zero_buf(zbuf, NSL)
    pltpu.sync_copy(zbuf, sumf_sh.at[pl.ds(s * NSL, NSL)])
    plsc.subcore_barrier()

    cf = cbuf[pl.ds(0, 16)][0]

    def _drain_a(n):
        def dr(j, _):
            pltpu.make_async_copy(abuf.at[pl.ds(0, 128)],
                                  sumf_sh.at[dst2d.at[0]], s0).wait()
            return 0
        lax.fori_loop(0, n, dr, 0)

    abase = s * EPT
    for ob in range(NB_A):
        b0 = abase + ob * OB
        pltpu.sync_copy(src_ref.at[pl.ds(b0, OB)], sbuf)
        pltpu.sync_copy(dst_ref.at[pl.ds(b0, OB)], dbuf)
        if ob > 0:
            _drain_a(OB // 128)

        def bodyA(i, _):
            sv = sbuf[pl.ds(i * 16, 16)]
            dv = dbuf[pl.ds(i * 16, 16)]
            x = (plsc.load_gather(fs_tab, [sv])
                 + plsc.load_gather(fd_tab, [dv]))
            wf = jnp.where(x >= 0, x, 0.01 * x)
            abuf[pl.ds(i * 16, 16)] = jnp.exp(wf - cf)
            _fill_dst2d(dst2d, i, dv)
            return 0

        lax.fori_loop(0, OB // 16, bodyA, 0)

        def bodyA2(j, _):
            pltpu.async_copy(abuf.at[pl.ds(j * 128, 128)],
                             sumf_sh.at[dst2d.at[j]], s0, add=True)
            return 0

        lax.fori_loop(0, OB // 128, bodyA2, 0)

    _drain_a(OB // 128)
    plsc.subcore_barrier()
    pltpu.sync_copy(sumf_sh, sumf_tab)

    bbase = s * EPT + c * EPW
    pltpu.sync_copy(src_ref.at[pl.ds(bbase, EPW)], sbuf_full)
    for ob in range(NB_B):
        b0 = bbase + ob * OB
        pltpu.sync_copy(dst_ref.at[pl.ds(b0, OB)], dbuf)
        pltpu.sync_copy(et_ref.at[pl.ds(b0, OB)], etbuf)

        def bodyB1(i, _, _ob=ob):
            sv = sbuf_full[pl.ds(_ob * OB + i * 16, 16)]
            dv = dbuf[pl.ds(i * 16, 16)]
            x = (plsc.load_gather(fs_tab, [sv])
                 + plsc.load_gather(fd_tab, [dv]))
            wf = jnp.where(x >= 0, x, 0.01 * x)
            ef = jnp.exp(wf - cf)
            sf = jnp.maximum(plsc.load_gather(sumf_tab, [dv]), 1e-30)
            st = jnp.maximum(plsc.load_gather(sumt_tab, [dv]), 1e-30)
            etv = etbuf[pl.ds(i * 16, 16)]
            abuf[pl.ds(_ob * OB + i * 16, 16)] = ETA * ef / sf + (1.0 - ETA) * etv / st
            row = _ob * 16 + i // 8
            col = (i % 8) * 16
            dst2d_full[row, pl.ds(col, 16)] = dv
            return 0

        lax.fori_loop(0, OB // 16, bodyB1, 0)

    NCH = EPW // 128
    zeros16 = jnp.zeros((16,), jnp.float32)
    for half, hin_ref in ((0, hq0_ref), (1, hq1_ref), (2, hq2_ref), (3, hq3_ref)):
        def zrows(r, _):
            for k in range(2):
                rows4[0, r, pl.ds(k * 16, 16)] = zeros16
            return 0

        lax.fori_loop(0, 128, zrows, 0)

        def zcopy(t, _):
            pltpu.sync_copy(rows4.at[0], z_sh.at[pl.ds(s * NSL + t * 128, 128)])
            return 0

        lax.fori_loop(0, NSL // 128, zcopy, 0)
        plsc.subcore_barrier()

        pltpu.async_copy(hin_ref.at[sbuf_full.at[pl.ds(0, 128)]], rows4.at[0], g0)
        pltpu.async_copy(hin_ref.at[sbuf_full.at[pl.ds(128, 128)]], rows4.at[1], g1)

        def chunk4(i, _, _hin=hin_ref):
            for b in range(4):
                t = 4 * i + b
                pltpu.make_async_copy(_hin.at[sbuf_full.at[pl.ds(0, 128)]],
                                      rows4.at[b], gsem[b]).wait()

                def scale(r2, _, _b=b, _t=t):
                    r = r2 * 2
                    a0 = abuf[pl.ds(_t * 128 + r, 16)][0]
                    a1 = abuf[pl.ds(_t * 128 + r + 1, 16)][0]
                    for k in range(2):
                        rows4[_b, r, pl.ds(k * 16, 16)] = (
                            rows4[_b, r, pl.ds(k * 16, 16)] * a0)
                        rows4[_b, r + 1, pl.ds(k * 16, 16)] = (
                            rows4[_b, r + 1, pl.ds(k * 16, 16)] * a1)
                    return 0

                lax.fori_loop(0, 64, scale, 0)
                pltpu.async_copy(rows4.at[b], z_sh.at[dst2d_full.at[t]],
                                 ssem[b], add=True)
                bp = (b + 2) % 4
                if b < 2:
                    @pl.when(i >= 1)
                    def _(_b=b, _bp=bp):
                        pltpu.make_async_copy(rows4.at[_bp],
                                              z_sh.at[dst2d_full.at[0]],
                                              ssem[_bp]).wait()
                    pltpu.async_copy(
                        _hin.at[sbuf_full.at[pl.ds((t + 2) * 128, 128)]],
                        rows4.at[bp], gsem[bp])
                else:
                    @pl.when(i < NCH // 4 - 1)
                    def _(_t=t, _bp=bp):
                        pltpu.make_async_copy(rows4.at[_bp],
                                              z_sh.at[dst2d_full.at[0]],
                                              ssem[_bp]).wait()
                        pltpu.async_copy(
                            _hin.at[sbuf_full.at[pl.ds((_t + 2) * 128, 128)]],
                            rows4.at[_bp], gsem[_bp])
            return 0

        lax.fori_loop(0, NCH // 4, chunk4, 0)
        pltpu.make_async_copy(rows4.at[0], z_sh.at[dst2d_full.at[0]], s0).wait()
        pltpu.make_async_copy(rows4.at[1], z_sh.at[dst2d_full.at[0]], s1).wait()
        pltpu.make_async_copy(rows4.at[2], z_sh.at[dst2d_full.at[0]], s2).wait()
        pltpu.make_async_copy(rows4.at[3], z_sh.at[dst2d_full.at[0]], s3).wait()
        plsc.subcore_barrier()

        def outb(t, _, _half=half):
            pltpu.sync_copy(z_sh.at[pl.ds(s * NSL + t * 128, 128)], rows4.at[0])
            pltpu.sync_copy(rows4.at[0], zout_ref.at[c, pl.ds(s * NSL + t * 128, 128),
                                                     pl.ds(_half * 32, 32)])
            return 0

        lax.fori_loop(0, NSL // 128, outb, 0)
        plsc.subcore_barrier()


def _sc_layer(src_p, dst_p, fsd, et, sumt_p, hq0, hq1, hq2, hq3, consts):
    f = pl.kernel(
        _sc_layer_body,
        out_type=_f32((NC, N_PAD, D)),
        mesh=_mesh,
        compiler_params=_SC_PARAMS,
        scratch_types=[
            pltpu.VMEM((N_PAD,), jnp.float32),
            pltpu.VMEM((N_PAD,), jnp.float32),
            pltpu.VMEM((N_PAD,), jnp.float32),
            pltpu.VMEM((N_PAD,), jnp.float32),
            pltpu.VMEM((EPW,), jnp.int32),
            pltpu.VMEM((OB,), jnp.int32),
            pltpu.VMEM((OB,), jnp.int32),
            pltpu.VMEM((OB,), jnp.float32),
            pltpu.VMEM((EPW + 16,), jnp.float32),
            pltpu.VMEM((16, 128), jnp.int32),
            pltpu.VMEM((NB_B * 16, 128), jnp.int32),
            pltpu.VMEM((4, 128, 32), jnp.float32),
            pltpu.VMEM((NSL,), jnp.float32),
            pltpu.VMEM((16,), jnp.float32),
            pltpu.SemaphoreType.DMA,
            pltpu.SemaphoreType.DMA,
            pltpu.SemaphoreType.DMA,
            pltpu.SemaphoreType.DMA,
            pltpu.SemaphoreType.DMA,
            pltpu.SemaphoreType.DMA,
            pltpu.SemaphoreType.DMA,
            pltpu.SemaphoreType.DMA,
            pltpu.VMEM_SHARED((N_PAD,), jnp.float32),
            pltpu.VMEM_SHARED((N_PAD, 32), jnp.float32),
        ],
    )
    return f(src_p, dst_p, fsd, et, sumt_p, hq0, hq1, hq2, hq3, consts)


def _sc_hrev_body(hin_ref, n2i_ref, hrevp_ref, ibuf2d, rows, hrev_sh):
    c = lax.axis_index("c")
    s = lax.axis_index("s")
    w = c * NS + s
    zeros16 = jnp.zeros((16,), jnp.float32)
    base = s * MSL
    for half in (0, 1, 2, 3):
        def zrows(r, _):
            for k in range(2):
                rows[r, pl.ds(k * 16, 16)] = zeros16
            return 0

        lax.fori_loop(0, 128, zrows, 0)
        pltpu.sync_copy(rows, hrev_sh.at[pl.ds(base, 128)])
        pltpu.sync_copy(rows, hrev_sh.at[pl.ds(base + 128, 128)])
        pltpu.sync_copy(rows.at[pl.ds(0, 64)], hrev_sh.at[pl.ds(base + 256, 64)])
        plsc.subcore_barrier()

        def chunk(ch, _half=half):
            r0 = ch * 128
            pltpu.sync_copy(n2i_ref.at[pl.ds(r0, 128)], ibuf2d.at[0])
            pltpu.sync_copy(hin_ref.at[pl.ds(r0, 128), pl.ds(_half * 32, 32)], rows)
            pltpu.sync_copy(rows, hrev_sh.at[ibuf2d.at[0]], add=True)

        chunk(w)
        chunk(w + 32)

        @pl.when(w < 16)
        def _():
            chunk(w + 64)

        plsc.subcore_barrier()
        for off, ln in ((0, 128), (128, 128), (256, 64)):
            pltpu.sync_copy(hrev_sh.at[pl.ds(base + off, ln)], rows.at[pl.ds(0, ln)])
            pltpu.sync_copy(rows.at[pl.ds(0, ln)],
                            hrevp_ref.at[c, pl.ds(base + off, ln), pl.ds(half * 32, 32)])
        plsc.subcore_barrier()


def _sc_hrev(hin, n2i_p):
    f = pl.kernel(
        _sc_hrev_body,
        out_type=_f32((NC, M_PAD, D)),
        mesh=_mesh,
        compiler_params=_SC_PARAMS,
        scratch_types=[
            pltpu.VMEM((1, 128), jnp.int32),
            pltpu.VMEM((128, 32), jnp.float32),
            pltpu.VMEM_SHARED((M_PAD, 32), jnp.float32),
        ],
    )
    return f(hin, n2i_p)


def _sc_gather_body(cur0_ref, cur1_ref, n2i_ref, out0_ref, out1_ref, ibuf, rows):
    c = lax.axis_index("c")
    s = lax.axis_index("s")
    w = c * NS + s
    nbase = w * (N_PAD // NW)
    pltpu.sync_copy(n2i_ref.at[pl.ds(nbase, 320)], ibuf)
    for tab_ref, out_ref in ((cur0_ref, out0_ref), (cur1_ref, out1_ref)):
        for off, ln in ((0, 128), (128, 128), (256, 64)):
            pltpu.sync_copy(tab_ref.at[ibuf.at[pl.ds(off, ln)]], rows.at[pl.ds(0, ln)])
            pltpu.sync_copy(rows.at[pl.ds(0, ln)], out_ref.at[pl.ds(nbase + off, ln)])


def _sc_gather(cur0, cur1, n2i_p):
    f = pl.kernel(
        _sc_gather_body,
        out_type=[_f32((N_PAD, D)), _f32((N_PAD, D))],
        mesh=_mesh,
        compiler_params=_SC_PARAMS,
        scratch_types=[
            pltpu.VMEM((320,), jnp.int32),
            pltpu.VMEM((128, D), jnp.float32),
        ],
    )
    return f(cur0, cur1, n2i_p)


def kernel(h, edge_index, taxo_cats, node2item, W_w, W_b, prelu_w, taxo_mean,
           taxo_std_log, wh_w, convW_w, convW_b, psi_w, psi_b, mlp0_w, mlp0_b,
           mlp1_w, mlp1_b, mlp2_w, mlp2_b):
    src_p = jnp.concatenate([edge_index[0], jnp.full((E_PAD - E,), DUMP, jnp.int32)])
    dst_p = jnp.concatenate([edge_index[1], jnp.full((E_PAD - E,), DUMP, jnp.int32)])
    n2i_p = jnp.concatenate([node2item.astype(jnp.int32),
                             jnp.full((N_PAD - N,), M, jnp.int32)])
    tc_flat = jnp.pad(taxo_cats.astype(jnp.int32), ((0, M_PAD - M), (0, 0))).reshape(-1)
    h_p = jnp.pad(h, ((0, N_PAD - N), (0, 0)))
    tm512 = jnp.pad(taxo_mean, ((0, TK - T), (0, 0)))

    h0q = _tc_pre(h_p, W_w, W_b, prelu_w, wh_w, tm512)
    h0q0, h0q1, h0q2, h0q3, fsd0, K, consts0 = h0q
    kflat = K.reshape(-1)
    consts0 = consts0.reshape(-1)

    c0n, c1n, tm0, tm1, sl0, sl1 = _sc_prep(n2i_p, tc_flat, taxo_mean, taxo_std_log)
    et, sumt_p, _mx = _sc_alphat(src_p, dst_p, c0n, c1n, kflat)

    z1 = _sc_layer(src_p, dst_p, fsd0, et, sumt_p, h0q0, h0q1, h0q2, h0q3, consts0)
    h1q0, h1q1, h1q2, h1q3, fsd1, consts1 = _tc_mid(z1, convW_w, convW_b,
                                                    prelu_w, wh_w)
    consts1 = consts1.reshape(-1)

    z2 = _sc_layer(src_p, dst_p, fsd1, et, sumt_p, h1q0, h1q1, h1q2, h1q3, consts1)
    raw = _tc_raw(z2, convW_w, convW_b)

    hrev_p = _sc_hrev(raw, n2i_p)
    cur0, cur1 = _tc_post(hrev_p, psi_w[0], psi_b, tm0, tm1, sl0, sl1)
    cur0n, cur1n = _sc_gather(cur0, cur1, n2i_p)
    eh, e0, e1 = _tc_emb(raw, cur0n, cur1n, mlp2_w, mlp2_b, mlp0_w, mlp0_b,
                         mlp1_w, mlp1_b)

    emb = jnp.concatenate(
        [eh[:N], jnp.stack([e0[:N], e1[:N]], axis=2).reshape(N, -1)], axis=1)
    return (raw[:N], emb)


# speedup vs baseline: 12.3716x; 1.0576x over previous
import functools

import jax
import jax.numpy as jnp
from jax import lax
from jax.experimental import pallas as pl
from jax.experimental.pallas import tpu as pltpu
from jax.experimental.pallas import tpu_sc as plsc

N = 10000
E = 320000
M = 5000
T = 500
D = 128
H = 128
ETA = 0.5

NC = 2
NS = 16
NW = NC * NS

N_PAD = 10240
M_PAD = 5120
E_PAD = 327680
TK = 512
DUMP = 10200

EPW = E_PAD // NW
EPT = E_PAD // NS
OB = 2048
NB_A = EPT // OB
NB_B = EPW // OB
NSL = N_PAD // NS
MSL = M_PAD // NS

_SC_PARAMS = pltpu.CompilerParams(needs_layout_passes=False,
                                  use_tc_tiling_on_sc=False)
_mesh = plsc.VectorSubcoreMesh(core_axis_name="c", subcore_axis_name="s",
                               num_cores=NC, num_subcores=NS)

_HIGH = jax.lax.Precision.HIGHEST


def _f32(shape):
    return jax.ShapeDtypeStruct(shape, jnp.float32)


def _i32(shape):
    return jax.ShapeDtypeStruct(shape, jnp.int32)


def _iota16():
    return lax.broadcasted_iota(jnp.int32, (16,), 0)


def _tc_pre_body(h_ref, W_ref, b_ref, pw_ref, wh_ref, tm512_ref,
                 h0q0_ref, h0q1_ref, h0q2_ref, h0q3_ref, fsd_ref, K_ref, consts_ref):
    x = jnp.dot(h_ref[...], W_ref[...].T, precision=_HIGH,
                preferred_element_type=jnp.float32) + b_ref[...]
    pw = pw_ref[0, 0]
    h0 = jnp.where(x >= 0, x, pw * x)
    h0q0_ref[...] = h0[:, 0:32]
    h0q1_ref[...] = h0[:, 32:64]
    h0q2_ref[...] = h0[:, 64:96]
    h0q3_ref[...] = h0[:, 96:128]
    wh = wh_ref[...]
    fs = jnp.dot(h0, wh[0, :H], precision=_HIGH, preferred_element_type=jnp.float32)
    fd = jnp.dot(h0, wh[0, H:], precision=_HIGH, preferred_element_type=jnp.float32)
    fsd_ref[0, :] = fs
    fsd_ref[1, :] = fd
    K = jnp.dot(tm512_ref[...], tm512_ref[...].T, precision=_HIGH,
                preferred_element_type=jnp.float32)
    K_ref[...] = K
    cmax = jnp.max(fs) + jnp.max(fd)
    cf = jnp.where(cmax >= 0, cmax, 0.01 * cmax)
    ct = 2.0 * jnp.max(K)
    col = lax.broadcasted_iota(jnp.int32, (1, 128), 1)
    consts_ref[...] = jnp.where(col == 0, cf, jnp.where(col == 1, ct, 0.0))


def _tc_pre(h_p, W_w, W_b, prelu_w, wh_w, tm512):
    return pl.pallas_call(
        _tc_pre_body,
        out_shape=[_f32((N_PAD, 32))] * 4 + [_f32((2, N_PAD)),
                   _f32((TK, TK)), _f32((1, 128))],
    )(h_p, W_w, W_b.reshape(1, H), prelu_w.reshape(1, 1), wh_w, tm512)


def _tc_mid_body(z_ref, convW_ref, b_ref, pw_ref, wh_ref,
                 h1q0_ref, h1q1_ref, h1q2_ref, h1q3_ref, fsd_ref, consts_ref):
    z = z_ref[0] + z_ref[1]
    x = jnp.dot(z, convW_ref[...].T, precision=_HIGH,
                preferred_element_type=jnp.float32) + b_ref[...]
    pw = pw_ref[0, 0]
    h1 = jnp.where(x >= 0, x, pw * x)
    h1q0_ref[...] = h1[:, 0:32]
    h1q1_ref[...] = h1[:, 32:64]
    h1q2_ref[...] = h1[:, 64:96]
    h1q3_ref[...] = h1[:, 96:128]
    wh = wh_ref[...]
    fs = jnp.dot(h1, wh[0, :H], precision=_HIGH, preferred_element_type=jnp.float32)
    fd = jnp.dot(h1, wh[0, H:], precision=_HIGH, preferred_element_type=jnp.float32)
    fsd_ref[0, :] = fs
    fsd_ref[1, :] = fd
    cmax = jnp.max(fs) + jnp.max(fd)
    cf = jnp.where(cmax >= 0, cmax, 0.01 * cmax)
    col = lax.broadcasted_iota(jnp.int32, (1, 128), 1)
    consts_ref[...] = jnp.where(col == 0, cf, 0.0)


def _tc_mid(z2, convW_w, convW_b, prelu_w, wh_w):
    return pl.pallas_call(
        _tc_mid_body,
        out_shape=[_f32((N_PAD, 32))] * 4 + [_f32((2, N_PAD)), _f32((1, 128))],
    )(z2, convW_w, convW_b.reshape(1, H), prelu_w.reshape(1, 1), wh_w)


def _tc_raw_body(z_ref, convW_ref, b_ref, raw_ref):
    z = z_ref[0] + z_ref[1]
    raw_ref[...] = jnp.dot(z, convW_ref[...].T, precision=_HIGH,
                           preferred_element_type=jnp.float32) + b_ref[...]


def _tc_raw(z2, convW_w, convW_b):
    return pl.pallas_call(
        _tc_raw_body, out_shape=_f32((N_PAD, D)),
    )(z2, convW_w, convW_b.reshape(1, H))


def _tc_post_body(hrev_ref, psi_ref, psib_ref, tm0_ref, tm1_ref, sl0_ref, sl1_ref,
                  cur0_ref, cur1_ref):
    hrev = hrev_ref[0] + hrev_ref[1]
    hp = jnp.dot(hrev, psi_ref[...], precision=_HIGH,
                 preferred_element_type=jnp.float32)
    pb = psib_ref[0, 0]
    for tm_ref, sl_ref, cur_ref in ((tm0_ref, sl0_ref, cur0_ref),
                                    (tm1_ref, sl1_ref, cur1_ref)):
        tm = tm_ref[...]
        stpl = jax.nn.sigmoid(jnp.sum(hp * tm, axis=1, keepdims=True) + pb)
        tau = stpl * jnp.exp(-jnp.exp(sl_ref[...]))
        cur_ref[...] = (1.0 - tau) * hrev + tau * tm


def _tc_post(hrev_p, psi0, psi_b, tm0, tm1, sl0, sl1):
    return pl.pallas_call(
        _tc_post_body,
        out_shape=[_f32((M_PAD, D)), _f32((M_PAD, D))],
    )(hrev_p, psi0, psi_b.reshape(1, 1), tm0, tm1, sl0, sl1)


def _l2norm(y):
    n = jnp.sqrt(jnp.sum(y * y, axis=1, keepdims=True))
    return y / jnp.maximum(n, 1e-12)


def _tc_emb_body(raw_ref, c0_ref, c1_ref, w2_ref, b2_ref, w0_ref, b0_ref,
                 w1_ref, b1_ref, eh_ref, e0_ref, e1_ref):
    eh_ref[...] = _l2norm(jnp.dot(raw_ref[...], w2_ref[...].T, precision=_HIGH,
                                  preferred_element_type=jnp.float32) + b2_ref[...])
    e0_ref[...] = _l2norm(jnp.dot(c0_ref[...], w0_ref[...].T, precision=_HIGH,
                                  preferred_element_type=jnp.float32) + b0_ref[...])
    e1_ref[...] = _l2norm(jnp.dot(c1_ref[...], w1_ref[...].T, precision=_HIGH,
                                  preferred_element_type=jnp.float32) + b1_ref[...])


def _tc_emb(raw, cur0n, cur1n, mlp2_w, mlp2_b, mlp0_w, mlp0_b, mlp1_w, mlp1_b):
    k = mlp2_w.shape[0]
    return pl.pallas_call(
        _tc_emb_body,
        out_shape=[_f32((N_PAD, k))] * 3,
        compiler_params=pltpu.CompilerParams(vmem_limit_bytes=100 * 1024 * 1024),
    )(raw, cur0n, cur1n, mlp2_w, mlp2_b.reshape(1, k), mlp0_w, mlp0_b.reshape(1, k),
      mlp1_w, mlp1_b.reshape(1, k))


def _wid():
    return lax.axis_index("c") * NS + lax.axis_index("s")


def _zero_buf(buf, n):
    zeros = jnp.zeros((16,), jnp.float32)

    def body(i, _):
        buf[pl.ds(i * 16, 16)] = zeros
        return 0

    lax.fori_loop(0, n // 16, body, 0)


def _sc_prep_body(n2i_ref, tc_ref, tm_ref, sl_ref,
                  c0n_ref, c1n_ref, tm0_ref, tm1_ref, sl0_ref, sl1_ref,
                  tc_tab, nbuf, c0buf, c1buf, ibuf, rows):
    c = lax.axis_index("c")
    s = lax.axis_index("s")
    w = c * NS + s
    pltpu.sync_copy(tc_ref, tc_tab)
    nbase = w * (N_PAD // NW)
    pltpu.sync_copy(n2i_ref.at[pl.ds(nbase, 320)], nbuf)

    def nbody(i, _):
        it = nbuf[pl.ds(i * 16, 16)]
        c0buf[pl.ds(i * 16, 16)] = plsc.load_gather(tc_tab, [it * 2])
        c1buf[pl.ds(i * 16, 16)] = plsc.load_gather(tc_tab, [it * 2 + 1])
        return 0

    lax.fori_loop(0, 20, nbody, 0)
    pltpu.sync_copy(c0buf, c0n_ref.at[pl.ds(nbase, 320)])
    pltpu.sync_copy(c1buf, c1n_ref.at[pl.ds(nbase, 320)])

    mbase = w * (M_PAD // NW)
    iota = _iota16()
    for lidx, outs in ((0, (tm0_ref, sl0_ref)), (1, (tm1_ref, sl1_ref))):
        def ibody(i, _, _l=lidx):
            item = mbase + i * 16 + iota
            ibuf[pl.ds(i * 16, 16)] = plsc.load_gather(tc_tab, [item * 2 + _l])
            return 0

        lax.fori_loop(0, 10, ibody, 0)
        for tab_ref, out_ref in ((tm_ref, outs[0]), (sl_ref, outs[1])):
            pltpu.sync_copy(tab_ref.at[ibuf.at[pl.ds(0, 128)]], rows.at[pl.ds(0, 128)])
            pltpu.sync_copy(tab_ref.at[ibuf.at[pl.ds(128, 32)]], rows.at[pl.ds(128, 32)])
            pltpu.sync_copy(rows, out_ref.at[pl.ds(mbase, 160)])


def _sc_prep(n2i_p, tc_flat, taxo_mean, taxo_std_log):
    f = pl.kernel(
        _sc_prep_body,
        out_type=[_i32((N_PAD,)), _i32((N_PAD,)),
                  _f32((M_PAD, D)), _f32((M_PAD, D)),
                  _f32((M_PAD, D)), _f32((M_PAD, D))],
        mesh=_mesh,
        compiler_params=_SC_PARAMS,
        scratch_types=[
            pltpu.VMEM((2 * M_PAD,), jnp.int32),
            pltpu.VMEM((320,), jnp.int32),
            pltpu.VMEM((320,), jnp.int32),
            pltpu.VMEM((320,), jnp.int32),
            pltpu.VMEM((160,), jnp.int32),
            pltpu.VMEM((160, D), jnp.float32),
        ],
    )
    return f(n2i_p, tc_flat, taxo_mean, taxo_std_log)


def _fill_dst2d(dst2d, i, dv):
    row = i // 8
    col = (i % 8) * 16
    dst2d[row, pl.ds(col, 16)] = dv


def _sc_alphat_body(src_ref, dst_ref, c0n_ref, c1n_ref, kf_ref,
                    et_ref, sumtp_ref, mxall_ref,
                    c0_tab, c1_tab, mx_tab, wt_store, sbuf, dbuf,
                    ka, kb, kva, kvb, etbuf, dst2d, zbuf, macc, mtmp,
                    ksem, tsem, msegm_sh, sumt_sh):
    c = lax.axis_index("c")
    s = lax.axis_index("s")
    pltpu.sync_copy(c0n_ref, c0_tab)
    pltpu.sync_copy(c1n_ref, c1_tab)
    neg = jnp.full((16,), -1e30, jnp.float32)

    def initm(i, _):
        mx_tab[pl.ds(i * 16, 16)] = neg
        return 0

    lax.fori_loop(0, N_PAD // 16, initm, 0)
    _zero_buf(zbuf, NSL)
    pltpu.sync_copy(zbuf, sumt_sh.at[pl.ds(s * NSL, NSL)])

    abase = s * EPT
    for ob in range(NB_A):
        b0 = abase + ob * OB
        pltpu.sync_copy(src_ref.at[pl.ds(b0, OB)], sbuf)
        pltpu.sync_copy(dst_ref.at[pl.ds(b0, OB)], dbuf)

        def body1(i, _):
            sv = sbuf[pl.ds(i * 16, 16)]
            dv = dbuf[pl.ds(i * 16, 16)]
            a0 = plsc.load_gather(c0_tab, [sv])
            b0v = plsc.load_gather(c0_tab, [dv])
            a1 = plsc.load_gather(c1_tab, [sv])
            b1v = plsc.load_gather(c1_tab, [dv])
            ka[pl.ds(i * 16, 16)] = a0 * TK + b0v
            kb[pl.ds(i * 16, 16)] = a1 * TK + b1v
            return 0

        lax.fori_loop(0, OB // 16, body1, 0)

        def body2(j, _):
            pltpu.async_copy(kf_ref.at[ka.at[pl.ds(j * 128, 128)]],
                             kva.at[pl.ds(j * 128, 128)], ksem)
            pltpu.async_copy(kf_ref.at[kb.at[pl.ds(j * 128, 128)]],
                             kvb.at[pl.ds(j * 128, 128)], ksem)
            return 0

        lax.fori_loop(0, OB // 128, body2, 0)

        def body2d(j, _):
            pltpu.make_async_copy(kf_ref.at[ka.at[pl.ds(0, 128)]],
                                  kva.at[pl.ds(0, 128)], ksem).wait()
            pltpu.make_async_copy(kf_ref.at[kb.at[pl.ds(0, 128)]],
                                  kvb.at[pl.ds(0, 128)], ksem).wait()
            return 0

        lax.fori_loop(0, OB // 128, body2d, 0)

        def body3(i, _, _ob=ob):
            dv = dbuf[pl.ds(i * 16, 16)]
            wt = kva[pl.ds(i * 16, 16)] + kvb[pl.ds(i * 16, 16)]
            wt_store[pl.ds(_ob * OB + i * 16, 16)] = wt
            cur = plsc.load_gather(mx_tab, [dv])
            need = wt > cur

            def mbody(m):
                plsc.store_scatter(mx_tab, [dv], wt, mask=m)
                cur2 = plsc.load_gather(mx_tab, [dv])
                return wt > cur2

            lax.while_loop(lambda m: jnp.any(m), mbody, need)
            return 0

        lax.fori_loop(0, OB // 16, body3, 0)

    pltpu.sync_copy(mx_tab, mxall_ref.at[c, s])
    plsc.subcore_barrier()
    pltpu.sync_copy(mxall_ref.at[c, 0, pl.ds(s * NSL, NSL)], macc)
    for t in range(1, NS):
        pltpu.sync_copy(mxall_ref.at[c, t, pl.ds(s * NSL, NSL)], mtmp)

        def mergeb(i, _):
            macc[pl.ds(i * 16, 16)] = jnp.maximum(macc[pl.ds(i * 16, 16)],
                                                  mtmp[pl.ds(i * 16, 16)])
            return 0

        lax.fori_loop(0, NSL // 16, mergeb, 0)
    pltpu.sync_copy(macc, msegm_sh.at[pl.ds(s * NSL, NSL)])
    plsc.subcore_barrier()
    pltpu.sync_copy(msegm_sh, mx_tab)

    bbase = s * EPT + c * EPW
    loff = c * EPW
    def _drain_t(n):
        def dr(j, _):
            pltpu.make_async_copy(etbuf.at[pl.ds(0, 128)],
                                  sumt_sh.at[dst2d.at[0]], tsem).wait()
            return 0
        lax.fori_loop(0, n, dr, 0)

    for ob in range(NB_B):
        b0 = bbase + ob * OB
        lo = loff + ob * OB
        pltpu.sync_copy(dst_ref.at[pl.ds(b0, OB)], dbuf)
        if ob > 0:
            _drain_t(OB // 128)

        def body4(i, _, _lo=lo):
            dv = dbuf[pl.ds(i * 16, 16)]
            ms = plsc.load_gather(mx_tab, [dv])
            wt = wt_store[pl.ds(_lo + i * 16, 16)]
            etv = jnp.exp(wt - ms)
            etbuf[pl.ds(i * 16, 16)] = etv
            _fill_dst2d(dst2d, i, dv)
            return 0

        lax.fori_loop(0, OB // 16, body4, 0)
        pltpu.sync_copy(etbuf, et_ref.at[pl.ds(b0, OB)])

        def body5(j, _):
            pltpu.async_copy(etbuf.at[pl.ds(j * 128, 128)],
                             sumt_sh.at[dst2d.at[j]], tsem, add=True)
            return 0

        lax.fori_loop(0, OB // 128, body5, 0)

    _drain_t(OB // 128)
    plsc.subcore_barrier()
    pltpu.sync_copy(sumt_sh.at[pl.ds(s * NSL, NSL)], zbuf)
    pltpu.sync_copy(zbuf, sumtp_ref.at[c, pl.ds(s * NSL, NSL)])


def _sc_alphat(src_p, dst_p, c0n, c1n, kflat):
    f = pl.kernel(
        _sc_alphat_body,
        out_type=[_f32((E_PAD,)), _f32((NC, N_PAD)), _f32((NC, NS, N_PAD))],
        mesh=_mesh,
        compiler_params=_SC_PARAMS,
        scratch_types=[
            pltpu.VMEM((N_PAD,), jnp.int32),
            pltpu.VMEM((N_PAD,), jnp.int32),
            pltpu.VMEM((N_PAD,), jnp.float32),
            pltpu.VMEM((EPT,), jnp.float32),
            pltpu.VMEM((OB,), jnp.int32),
            pltpu.VMEM((OB,), jnp.int32),
            pltpu.VMEM((OB,), jnp.int32),
            pltpu.VMEM((OB,), jnp.int32),
            pltpu.VMEM((OB,), jnp.float32),
            pltpu.VMEM((OB,), jnp.float32),
            pltpu.VMEM((OB,), jnp.float32),
            pltpu.VMEM((16, 128), jnp.int32),
            pltpu.VMEM((NSL,), jnp.float32),
            pltpu.VMEM((NSL,), jnp.float32),
            pltpu.VMEM((NSL,), jnp.float32),
            pltpu.SemaphoreType.DMA,
            pltpu.SemaphoreType.DMA,
            pltpu.VMEM_SHARED((N_PAD,), jnp.float32),
            pltpu.VMEM_SHARED((N_PAD,), jnp.float32),
        ],
    )
    return f(src_p, dst_p, c0n, c1n, kflat)


def _sc_layer_body(src_ref, dst_ref, fsd_ref, et_ref, sumtp_ref,
                   hq0_ref, hq1_ref, hq2_ref, hq3_ref, consts_ref, zout_ref,
                   fs_tab, fd_tab, sumf_tab, sumt_tab,
                   sbuf_full, sbuf, dbuf, etbuf, abuf, dst2d, dst2d_full,
                   rows4, zbuf, cbuf,
                   g0, g1, g2, g3, s0, s1, s2, s3,
                   sumf_sh, z_sh):
    c = lax.axis_index("c")
    s = lax.axis_index("s")
    gsem = (g0, g1, g2, g3)
    ssem = (s0, s1, s2, s3)
    pltpu.sync_copy(fsd_ref.at[0], fs_tab)
    pltpu.sync_copy(fsd_ref.at[1], fd_tab)
    pltpu.sync_copy(consts_ref.at[pl.ds(0, 16)], cbuf)
    pltpu.sync_copy(sumtp_ref.at[0], sumt_tab)
    pltpu.sync_copy(sumtp_ref.at[1], sumf_tab)

    def addb(i, _):
        sumt_tab[pl.ds(i * 16, 16)] = (sumt_tab[pl.ds(i * 16, 16)]
                                       + sumf_tab[pl.ds(i * 16, 16)])
        return 0

    lax.fori_loop(0, N_PAD // 16, addb, 0)
    _zero_buf(zbuf, NSL)
    pltpu.sync_copy(zbuf, sumf_sh.at[pl.ds(s * NSL, NSL)])
    plsc.subcore_barrier()

    cf = cbuf[pl.ds(0, 16)][0]

    def _drain_a(n):
        def dr(j, _):
            pltpu.make_async_copy(abuf.at[pl.ds(0, 128)],
                                  sumf_sh.at[dst2d.at[0]], s0).wait()
            return 0
        lax.fori_loop(0, n, dr, 0)

    abase = s * EPT
    for ob in range(NB_A):
        b0 = abase + ob * OB
        pltpu.sync_copy(src_ref.at[pl.ds(b0, OB)], sbuf)
        pltpu.sync_copy(dst_ref.at[pl.ds(b0, OB)], dbuf)
        if ob > 0:
            _drain_a(OB // 128)

        def bodyA(i, _):
            sv = sbuf[pl.ds(i * 16, 16)]
            dv = dbuf[pl.ds(i * 16, 16)]
            x = (plsc.load_gather(fs_tab, [sv])
                 + plsc.load_gather(fd_tab, [dv]))
            wf = jnp.where(x >= 0, x, 0.01 * x)
            abuf[pl.ds(i * 16, 16)] = jnp.exp(wf - cf)
            _fill_dst2d(dst2d, i, dv)
            return 0

        lax.fori_loop(0, OB // 16, bodyA, 0)

        def bodyA2(j, _):
            pltpu.async_copy(abuf.at[pl.ds(j * 128, 128)],
                             sumf_sh.at[dst2d.at[j]], s0, add=True)
            return 0

        lax.fori_loop(0, OB // 128, bodyA2, 0)

    _drain_a(OB // 128)
    plsc.subcore_barrier()
    pltpu.sync_copy(sumf_sh, sumf_tab)

    bbase = s * EPT + c * EPW
    pltpu.sync_copy(src_ref.at[pl.ds(bbase, EPW)], sbuf_full)
    for ob in range(NB_B):
        b0 = bbase + ob * OB
        pltpu.sync_copy(dst_ref.at[pl.ds(b0, OB)], dbuf)
        pltpu.sync_copy(et_ref.at[pl.ds(b0, OB)], etbuf)

        def bodyB1(i, _, _ob=ob):
            sv = sbuf_full[pl.ds(_ob * OB + i * 16, 16)]
            dv = dbuf[pl.ds(i * 16, 16)]
            x = (plsc.load_gather(fs_tab, [sv])
                 + plsc.load_gather(fd_tab, [dv]))
            wf = jnp.where(x >= 0, x, 0.01 * x)
            ef = jnp.exp(wf - cf)
            sf = jnp.maximum(plsc.load_gather(sumf_tab, [dv]), 1e-30)
            st = jnp.maximum(plsc.load_gather(sumt_tab, [dv]), 1e-30)
            etv = etbuf[pl.ds(i * 16, 16)]
            abuf[pl.ds(_ob * OB + i * 16, 16)] = ETA * ef / sf + (1.0 - ETA) * etv / st
            row = _ob * 16 + i // 8
            col = (i % 8) * 16
            dst2d_full[row, pl.ds(col, 16)] = dv
            return 0

        lax.fori_loop(0, OB // 16, bodyB1, 0)

    NCH = EPW // 128
    zeros16 = jnp.zeros((16,), jnp.float32)
    for half, hin_ref in ((0, hq0_ref), (1, hq1_ref), (2, hq2_ref), (3, hq3_ref)):
        def zrows(r, _):
            for k in range(2):
                rows4[0, r, pl.ds(k * 16, 16)] = zeros16
            return 0

        lax.fori_loop(0, 128, zrows, 0)

        def zcopy(t, _):
            pltpu.sync_copy(rows4.at[0], z_sh.at[pl.ds(s * NSL + t * 128, 128)])
            return 0

        lax.fori_loop(0, NSL // 128, zcopy, 0)
        plsc.subcore_barrier()

        pltpu.async_copy(hin_ref.at[sbuf_full.at[pl.ds(0, 128)]], rows4.at[0], g0)
        pltpu.async_copy(hin_ref.at[sbuf_full.at[pl.ds(128, 128)]], rows4.at[1], g1)

        def chunk4(i, _, _hin=hin_ref):
            for b in range(4):
                t = 4 * i + b
                pltpu.make_async_copy(_hin.at[sbuf_full.at[pl.ds(0, 128)]],
                                      rows4.at[b], gsem[b]).wait()

                def scale(r4, _, _b=b, _t=t):
                    r = r4 * 4
                    for u in range(4):
                        a = abuf[pl.ds(_t * 128 + r + u, 16)][0]
                        for k in range(2):
                            rows4[_b, r + u, pl.ds(k * 16, 16)] = (
                                rows4[_b, r + u, pl.ds(k * 16, 16)] * a)
                    return 0

                lax.fori_loop(0, 32, scale, 0)
                pltpu.async_copy(rows4.at[b], z_sh.at[dst2d_full.at[t]],
                                 ssem[b], add=True)
                bp = (b + 2) % 4
                if b < 2:
                    @pl.when(i >= 1)
                    def _(_b=b, _bp=bp):
                        pltpu.make_async_copy(rows4.at[_bp],
                                              z_sh.at[dst2d_full.at[0]],
                                              ssem[_bp]).wait()
                    pltpu.async_copy(
                        _hin.at[sbuf_full.at[pl.ds((t + 2) * 128, 128)]],
                        rows4.at[bp], gsem[bp])
                else:
                    @pl.when(i < NCH // 4 - 1)
                    def _(_t=t, _bp=bp):
                        pltpu.make_async_copy(rows4.at[_bp],
                                              z_sh.at[dst2d_full.at[0]],
                                              ssem[_bp]).wait()
                        pltpu.async_copy(
                            _hin.at[sbuf_full.at[pl.ds((_t + 2) * 128, 128)]],
                            rows4.at[_bp], gsem[_bp])
            return 0

        lax.fori_loop(0, NCH // 4, chunk4, 0)
        pltpu.make_async_copy(rows4.at[0], z_sh.at[dst2d_full.at[0]], s0).wait()
        pltpu.make_async_copy(rows4.at[1], z_sh.at[dst2d_full.at[0]], s1).wait()
        pltpu.make_async_copy(rows4.at[2], z_sh.at[dst2d_full.at[0]], s2).wait()
        pltpu.make_async_copy(rows4.at[3], z_sh.at[dst2d_full.at[0]], s3).wait()
        plsc.subcore_barrier()

        def outb(t, _, _half=half):
            pltpu.sync_copy(z_sh.at[pl.ds(s * NSL + t * 128, 128)], rows4.at[0])
            pltpu.sync_copy(rows4.at[0], zout_ref.at[c, pl.ds(s * NSL + t * 128, 128),
                                                     pl.ds(_half * 32, 32)])
            return 0

        lax.fori_loop(0, NSL // 128, outb, 0)
        plsc.subcore_barrier()


def _sc_layer(src_p, dst_p, fsd, et, sumt_p, hq0, hq1, hq2, hq3, consts):
    f = pl.kernel(
        _sc_layer_body,
        out_type=_f32((NC, N_PAD, D)),
        mesh=_mesh,
        compiler_params=_SC_PARAMS,
        scratch_types=[
            pltpu.VMEM((N_PAD,), jnp.float32),
            pltpu.VMEM((N_PAD,), jnp.float32),
            pltpu.VMEM((N_PAD,), jnp.float32),
            pltpu.VMEM((N_PAD,), jnp.float32),
            pltpu.VMEM((EPW,), jnp.int32),
            pltpu.VMEM((OB,), jnp.int32),
            pltpu.VMEM((OB,), jnp.int32),
            pltpu.VMEM((OB,), jnp.float32),
            pltpu.VMEM((EPW + 16,), jnp.float32),
            pltpu.VMEM((16, 128), jnp.int32),
            pltpu.VMEM((NB_B * 16, 128), jnp.int32),
            pltpu.VMEM((4, 128, 32), jnp.float32),
            pltpu.VMEM((NSL,), jnp.float32),
            pltpu.VMEM((16,), jnp.float32),
            pltpu.SemaphoreType.DMA,
            pltpu.SemaphoreType.DMA,
            pltpu.SemaphoreType.DMA,
            pltpu.SemaphoreType.DMA,
            pltpu.SemaphoreType.DMA,
            pltpu.SemaphoreType.DMA,
            pltpu.SemaphoreType.DMA,
            pltpu.SemaphoreType.DMA,
            pltpu.VMEM_SHARED((N_PAD,), jnp.float32),
            pltpu.VMEM_SHARED((N_PAD, 32), jnp.float32),
        ],
    )
    return f(src_p, dst_p, fsd, et, sumt_p, hq0, hq1, hq2, hq3, consts)


def _sc_hrev_body(hin_ref, n2i_ref, hrevp_ref, ibuf2d, rows, hrev_sh):
    c = lax.axis_index("c")
    s = lax.axis_index("s")
    w = c * NS + s
    zeros16 = jnp.zeros((16,), jnp.float32)
    base = s * MSL
    for half in (0, 1, 2, 3):
        def zrows(r, _):
            for k in range(2):
                rows[r, pl.ds(k * 16, 16)] = zeros16
            return 0

        lax.fori_loop(0, 128, zrows, 0)
        pltpu.sync_copy(rows, hrev_sh.at[pl.ds(base, 128)])
        pltpu.sync_copy(rows, hrev_sh.at[pl.ds(base + 128, 128)])
        pltpu.sync_copy(rows.at[pl.ds(0, 64)], hrev_sh.at[pl.ds(base + 256, 64)])
        plsc.subcore_barrier()

        def chunk(ch, _half=half):
            r0 = ch * 128
            pltpu.sync_copy(n2i_ref.at[pl.ds(r0, 128)], ibuf2d.at[0])
            pltpu.sync_copy(hin_ref.at[pl.ds(r0, 128), pl.ds(_half * 32, 32)], rows)
            pltpu.sync_copy(rows, hrev_sh.at[ibuf2d.at[0]], add=True)

        chunk(w)
        chunk(w + 32)

        @pl.when(w < 16)
        def _():
            chunk(w + 64)

        plsc.subcore_barrier()
        for off, ln in ((0, 128), (128, 128), (256, 64)):
            pltpu.sync_copy(hrev_sh.at[pl.ds(base + off, ln)], rows.at[pl.ds(0, ln)])
            pltpu.sync_copy(rows.at[pl.ds(0, ln)],
                            hrevp_ref.at[c, pl.ds(base + off, ln), pl.ds(half * 32, 32)])
        plsc.subcore_barrier()


def _sc_hrev(hin, n2i_p):
    f = pl.kernel(
        _sc_hrev_body,
        out_type=_f32((NC, M_PAD, D)),
        mesh=_mesh,
        compiler_params=_SC_PARAMS,
        scratch_types=[
            pltpu.VMEM((1, 128), jnp.int32),
            pltpu.VMEM((128, 32), jnp.float32),
            pltpu.VMEM_SHARED((M_PAD, 32), jnp.float32),
        ],
    )
    return f(hin, n2i_p)


def _sc_gather_body(cur0_ref, cur1_ref, n2i_ref, out0_ref, out1_ref, ibuf, rows):
    c = lax.axis_index("c")
    s = lax.axis_index("s")
    w = c * NS + s
    nbase = w * (N_PAD // NW)
    pltpu.sync_copy(n2i_ref.at[pl.ds(nbase, 320)], ibuf)
    for tab_ref, out_ref in ((cur0_ref, out0_ref), (cur1_ref, out1_ref)):
        for off, ln in ((0, 128), (128, 128), (256, 64)):
            pltpu.sync_copy(tab_ref.at[ibuf.at[pl.ds(off, ln)]], rows.at[pl.ds(0, ln)])
            pltpu.sync_copy(rows.at[pl.ds(0, ln)], out_ref.at[pl.ds(nbase + off, ln)])


def _sc_gather(cur0, cur1, n2i_p):
    f = pl.kernel(
        _sc_gather_body,
        out_type=[_f32((N_PAD, D)), _f32((N_PAD, D))],
        mesh=_mesh,
        compiler_params=_SC_PARAMS,
        scratch_types=[
            pltpu.VMEM((320,), jnp.int32),
            pltpu.VMEM((128, D), jnp.float32),
        ],
    )
    return f(cur0, cur1, n2i_p)


def kernel(h, edge_index, taxo_cats, node2item, W_w, W_b, prelu_w, taxo_mean,
           taxo_std_log, wh_w, convW_w, convW_b, psi_w, psi_b, mlp0_w, mlp0_b,
           mlp1_w, mlp1_b, mlp2_w, mlp2_b):
    src_p = jnp.concatenate([edge_index[0], jnp.full((E_PAD - E,), DUMP, jnp.int32)])
    dst_p = jnp.concatenate([edge_index[1], jnp.full((E_PAD - E,), DUMP, jnp.int32)])
    n2i_p = jnp.concatenate([node2item.astype(jnp.int32),
                             jnp.full((N_PAD - N,), M, jnp.int32)])
    tc_flat = jnp.pad(taxo_cats.astype(jnp.int32), ((0, M_PAD - M), (0, 0))).reshape(-1)
    h_p = jnp.pad(h, ((0, N_PAD - N), (0, 0)))
    tm512 = jnp.pad(taxo_mean, ((0, TK - T), (0, 0)))

    h0q = _tc_pre(h_p, W_w, W_b, prelu_w, wh_w, tm512)
    h0q0, h0q1, h0q2, h0q3, fsd0, K, consts0 = h0q
    kflat = K.reshape(-1)
    consts0 = consts0.reshape(-1)

    c0n, c1n, tm0, tm1, sl0, sl1 = _sc_prep(n2i_p, tc_flat, taxo_mean, taxo_std_log)
    et, sumt_p, _mx = _sc_alphat(src_p, dst_p, c0n, c1n, kflat)

    z1 = _sc_layer(src_p, dst_p, fsd0, et, sumt_p, h0q0, h0q1, h0q2, h0q3, consts0)
    h1q0, h1q1, h1q2, h1q3, fsd1, consts1 = _tc_mid(z1, convW_w, convW_b,
                                                    prelu_w, wh_w)
    consts1 = consts1.reshape(-1)

    z2 = _sc_layer(src_p, dst_p, fsd1, et, sumt_p, h1q0, h1q1, h1q2, h1q3, consts1)
    raw = _tc_raw(z2, convW_w, convW_b)

    hrev_p = _sc_hrev(raw, n2i_p)
    cur0, cur1 = _tc_post(hrev_p, psi_w[0], psi_b, tm0, tm1, sl0, sl1)
    cur0n, cur1n = _sc_gather(cur0, cur1, n2i_p)
    eh, e0, e1 = _tc_emb(raw, cur0n, cur1n, mlp2_w, mlp2_b, mlp0_w, mlp0_b,
                         mlp1_w, mlp1_b)

    emb = jnp.concatenate(
        [eh[:N], jnp.stack([e0[:N], e1[:N]], axis=2).reshape(N, -1)], axis=1)
    return (raw[:N], emb)
